# Initial kernel scaffold; baseline (speedup 1.0000x reference)
import jax
import jax.numpy as jnp
from jax.experimental import pallas as pl


def kernel(idx_x, x_batch, idx_w, edge_w, edge_id, edge_id_batch, edge_index, whole_edge, word_vec, word_vec_beta, topic_vec, topic_edge_vec, W_topic, W_enc, b_enc, W_phi, b_phi, W_mu, b_mu, W_lv, b_lv):
    raise NotImplementedError("write your pallas kernel here")


# speedup vs baseline: 2.9002x; 2.9002x over previous
import functools
import jax
import jax.numpy as jnp
from jax import lax
from jax.experimental import pallas as pl
from jax.experimental.pallas import tpu as pltpu
from jax.experimental.pallas import tpu_sc as plsc

N = 4096
E = 65536
V = 50000
NI = 128
K = 50
KP = 128
G = 64
EW = 100000
VP = 50176
EWP = 100352
TEMP = 0.5

NC = 2
NS = 16
NW = NC * NS
EPW = E // NW
NPW = N // NW
WPW = EWP // NW

def _wid():
    return lax.axis_index("s") * NC + lax.axis_index("c")


_SC1_TYPES = dict(
    out_type=[
        jax.ShapeDtypeStruct((N, NI), jnp.float32),
        jax.ShapeDtypeStruct((N, NI), jnp.float32),
        jax.ShapeDtypeStruct((E, NI), jnp.float32),
        jax.ShapeDtypeStruct((E,), jnp.int32),
        jax.ShapeDtypeStruct((E,), jnp.int32),
        jax.ShapeDtypeStruct((E,), jnp.int32),
        jax.ShapeDtypeStruct((E,), jnp.float32),
    ],
    scratch_types=[
        pltpu.VMEM((N,), jnp.int32),
        pltpu.VMEM((N,), jnp.int32),
        pltpu.VMEM((NPW,), jnp.int32),
        pltpu.VMEM((512, NI), jnp.float32),
        pltpu.VMEM((EPW,), jnp.int32),
        pltpu.VMEM((EPW,), jnp.int32),
        pltpu.VMEM((EPW,), jnp.int32),
        pltpu.VMEM((EPW,), jnp.int32),
        pltpu.VMEM((EPW,), jnp.int32),
        pltpu.VMEM((EPW,), jnp.int32),
        pltpu.VMEM((EPW,), jnp.float32),
        pltpu.SemaphoreType.DMA,
    ],
)


def _sc1(wv_h, wvb_h, idxx_h, e0_h, skey_h, xb_h,
         h0_o, h0b_o, h0e_o, ui_o, uj_o, gi_o, val_o,
         idxx_v, xb_v, idxn_v, rowbuf, e0_v, idx2_v, sk_v,
         ui_v, uj_v, gi_v, val_v, sem):
    wid = _wid()
    base_n = wid * NPW
    base_e = wid * EPW
    pltpu.sync_copy(idxx_h.at[pl.ds(base_n, NPW)], idxn_v)
    pltpu.async_copy(wv_h.at[idxn_v], rowbuf.at[pl.ds(0, NPW)], sem).wait()
    pltpu.sync_copy(rowbuf.at[pl.ds(0, NPW)], h0_o.at[pl.ds(base_n, NPW)])
    pltpu.async_copy(wvb_h.at[idxn_v], rowbuf.at[pl.ds(0, NPW)], sem).wait()
    pltpu.sync_copy(rowbuf.at[pl.ds(0, NPW)], h0b_o.at[pl.ds(base_n, NPW)])
    pltpu.sync_copy(idxx_h, idxx_v)
    pltpu.sync_copy(xb_h, xb_v)
    pltpu.sync_copy(e0_h.at[pl.ds(base_e, EPW)], e0_v)

    def body_i2(i, _):
        ev = e0_v[pl.ds(i * 16, 16)]
        idx2_v[pl.ds(i * 16, 16)] = plsc.load_gather(idxx_v, [ev])
        return 0
    lax.fori_loop(0, EPW // 16, body_i2, 0)
    for c in range(EPW // 512):
        pltpu.async_copy(wv_h.at[idx2_v.at[pl.ds(c * 512, 512)]], rowbuf,
                         sem).wait()
        pltpu.sync_copy(rowbuf, h0e_o.at[pl.ds(base_e + c * 512, 512)])
    pltpu.sync_copy(skey_h.at[pl.ds(base_e, EPW)], sk_v)

    def body_uk(i, _):
        sl = pl.ds(i * 16, 16)
        kv = sk_v[sl]
        uiv = lax.shift_right_logical(kv, 12)
        ujv = lax.bitwise_and(kv, 4095)
        giv = plsc.load_gather(xb_v, [uiv])
        gjv = plsc.load_gather(xb_v, [ujv])
        ui_v[sl] = uiv
        uj_v[sl] = ujv
        gi_v[sl] = giv
        ok = jnp.logical_and(giv == gjv, uiv != ujv)
        val_v[sl] = jnp.where(ok, 1.0, 0.0).astype(jnp.float32)
        return 0
    lax.fori_loop(0, EPW // 16, body_uk, 0)
    pltpu.sync_copy(ui_v, ui_o.at[pl.ds(base_e, EPW)])
    pltpu.sync_copy(uj_v, uj_o.at[pl.ds(base_e, EPW)])
    pltpu.sync_copy(gi_v, gi_o.at[pl.ds(base_e, EPW)])
    pltpu.sync_copy(val_v, val_o.at[pl.ds(base_e, EPW)])


_SC2_TYPES = dict(
    out_type=[
        jax.ShapeDtypeStruct((E, KP), jnp.float32),
        jax.ShapeDtypeStruct((E, KP), jnp.float32),
        jax.ShapeDtypeStruct((E, KP), jnp.float32),
        jax.ShapeDtypeStruct((E, KP), jnp.float32),
        jax.ShapeDtypeStruct((EWP, KP), jnp.float32),
        jax.ShapeDtypeStruct((EWP, KP), jnp.float32),
    ],
    scratch_types=[
        pltpu.VMEM((EPW,), jnp.int32),
        pltpu.VMEM((WPW,), jnp.int32),
        pltpu.VMEM((512, KP), jnp.float32),
        pltpu.SemaphoreType.DMA,
    ],
)


def _sc2(phi_h, qp_h, qn_h, at_h, bt_h, e0_h, e1_h, ui_h, uj_h,
         we0_h, we1_h,
         phi1_o, qpe0_o, qnui_o, phiuj_o, a100_o, b100_o,
         idx_v, bigidx_v, rowbuf, sem):
    wid = _wid()
    base_e = wid * EPW
    base_w = wid * WPW
    for idx_h, tab_h, out_o in ((e1_h, phi_h, phi1_o), (e0_h, qp_h, qpe0_o),
                                (ui_h, qn_h, qnui_o), (uj_h, phi_h, phiuj_o)):
        pltpu.sync_copy(idx_h.at[pl.ds(base_e, EPW)], idx_v)
        for c in range(EPW // 512):
            pltpu.async_copy(tab_h.at[idx_v.at[pl.ds(c * 512, 512)]], rowbuf,
                             sem).wait()
            pltpu.sync_copy(rowbuf, out_o.at[pl.ds(base_e + c * 512, 512)])
    for widx_h, tab_h, out_o in ((we0_h, at_h, a100_o), (we1_h, bt_h, b100_o)):
        pltpu.sync_copy(widx_h.at[pl.ds(base_w, WPW)], bigidx_v)
        for c in range(WPW // 448):
            pltpu.async_copy(tab_h.at[bigidx_v.at[pl.ds(c * 448, 448)]],
                             rowbuf.at[pl.ds(0, 448)], sem).wait()
            pltpu.sync_copy(rowbuf.at[pl.ds(0, 448)],
                            out_o.at[pl.ds(base_w + c * 448, 448)])


_SC3_TYPES = dict(
    out_type=[
        jax.ShapeDtypeStruct((E,), jnp.float32),
        jax.ShapeDtypeStruct((E,), jnp.int32),
        jax.ShapeDtypeStruct((E,), jnp.float32),
    ],
    scratch_types=[
        pltpu.VMEM((N,), jnp.int32),
        pltpu.VMEM((EPW,), jnp.int32),
        pltpu.VMEM((EPW,), jnp.int32),
        pltpu.VMEM((EPW,), jnp.int32),
        pltpu.VMEM((EPW,), jnp.int32),
        pltpu.VMEM((EPW,), jnp.float32),
        pltpu.VMEM((EPW,), jnp.float32),
        pltpu.VMEM((512, KP), jnp.float32),
        pltpu.SemaphoreType.DMA,
    ],
)


def _sc3(a100_h, b100_h, eid_h, kz_h, e0_h, e1_h,
         sval_o, kz0_o, same_o,
         kz_v, eid_v, e0_v, e1_v, kz0_v, sv_v, same_v, rowbuf, sem):
    wid = _wid()
    base_e = wid * EPW
    pltpu.sync_copy(kz_h, kz_v)
    pltpu.sync_copy(eid_h.at[pl.ds(base_e, EPW)], eid_v)
    pltpu.sync_copy(e0_h.at[pl.ds(base_e, EPW)], e0_v)
    pltpu.sync_copy(e1_h.at[pl.ds(base_e, EPW)], e1_v)

    def body_kz(i, _):
        sl = pl.ds(i * 16, 16)
        k0 = plsc.load_gather(kz_v, [e0_v[sl]])
        k1 = plsc.load_gather(kz_v, [e1_v[sl]])
        kz0_v[sl] = k0
        same_v[sl] = jnp.where(k0 == k1, 1.0, 0.0).astype(jnp.float32)
        return 0
    lax.fori_loop(0, EPW // 16, body_kz, 0)

    rows16 = lax.iota(jnp.int32, 16)
    for c in range(EPW // 512):
        pltpu.async_copy(a100_h.at[eid_v.at[pl.ds(c * 512, 512)]], rowbuf,
                         sem).wait()

        def body_pa(j, _):
            sl = pl.ds(c * 512 + j * 16, 16)
            va = plsc.load_gather(rowbuf, [rows16 + j * 16, kz0_v[sl]])
            sv_v[sl] = va
            return 0
        lax.fori_loop(0, 512 // 16, body_pa, 0)
        pltpu.async_copy(b100_h.at[eid_v.at[pl.ds(c * 512, 512)]], rowbuf,
                         sem).wait()

        def body_pb(j, _):
            sl = pl.ds(c * 512 + j * 16, 16)
            vb = plsc.load_gather(rowbuf, [rows16 + j * 16, kz0_v[sl]])
            sv_v[sl] = sv_v[sl] + vb
            return 0
        lax.fori_loop(0, 512 // 16, body_pb, 0)
    pltpu.sync_copy(sv_v, sval_o.at[pl.ds(base_e, EPW)])
    pltpu.sync_copy(kz0_v, kz0_o.at[pl.ds(base_e, EPW)])
    pltpu.sync_copy(same_v, same_o.at[pl.ds(base_e, EPW)])


@functools.lru_cache(maxsize=1)
def _sc_kernels():
    mesh = plsc.VectorSubcoreMesh(core_axis_name="c", subcore_axis_name="s")
    cp = pltpu.CompilerParams(needs_layout_passes=False)
    sc1 = pl.kernel(_sc1, mesh=mesh, compiler_params=cp, **_SC1_TYPES)
    sc2 = pl.kernel(_sc2, mesh=mesh, compiler_params=cp, **_SC2_TYPES)
    sc3 = pl.kernel(_sc3, mesh=mesh, compiler_params=cp, **_SC3_TYPES)
    return sc1, sc2, sc3


def _tc_a(tvcat_ref, wt_ref, lpw_ref, lnw_ref):
    tv = jnp.dot(tvcat_ref[...], wt_ref[...].T,
                 preferred_element_type=jnp.float32)
    s = jnp.dot(tv, tv.T, preferred_element_type=jnp.float32)
    wm = jnp.clip(jax.nn.sigmoid(s), 1e-6, 1.0 - 1e-6)
    r = lax.broadcasted_iota(jnp.int32, (KP, KP), 0)
    c = lax.broadcasted_iota(jnp.int32, (KP, KP), 1)
    mask = jnp.where(jnp.logical_and(r < K, c < K), 1.0, 0.0)
    lpw_ref[...] = jnp.log(wm) * mask
    lnw_ref[...] = jnp.log(1.0 - wm) * mask


def _tc_b(e1_ref, h0e_ref, ew_ref, msg_ref):
    pid = pl.program_id(0)

    @pl.when(pid == 0)
    def _():
        msg_ref[...] = jnp.zeros_like(msg_ref)
    e1b = e1_ref[0]
    iota_n = lax.broadcasted_iota(jnp.int32, (N, e1b.shape[1]), 0)
    onehot = jnp.where(e1b == iota_n, 1.0, 0.0) * ew_ref[0]
    msg_ref[...] += jnp.dot(onehot, h0e_ref[...],
                            preferred_element_type=jnp.float32)


def _tc_c(h0_ref, msg_ref, wenc_ref, benc_ref, wphi_ref, bphi_ref,
          lpw_ref, lnw_ref, gumb_ref,
          h_ref, phi_ref, qp_ref, qn_ref, kz_ref):
    x = h0_ref[...] + msg_ref[...]
    h = jax.nn.relu(jnp.dot(x, wenc_ref[...],
                            preferred_element_type=jnp.float32) + benc_ref[...])
    h_ref[...] = h
    logits = jnp.dot(h, wphi_ref[...],
                     preferred_element_type=jnp.float32) + bphi_ref[...]
    m = jnp.max(logits, axis=1, keepdims=True)
    ex = jnp.exp(logits - m)
    phi = ex / jnp.sum(ex, axis=1, keepdims=True)
    phi_ref[...] = phi
    qp_ref[...] = jnp.dot(phi, lpw_ref[...], preferred_element_type=jnp.float32)
    qn_ref[...] = jnp.dot(phi, lnw_ref[...], preferred_element_type=jnp.float32)
    gl = jnp.log(phi + 1e-20) + gumb_ref[...]
    gm = jnp.max(gl, axis=1, keepdims=True)
    iota_k = lax.broadcasted_iota(jnp.int32, gl.shape, 1)
    cand = jnp.where(gl >= gm, iota_k, jnp.int32(10**9))
    kz = jnp.min(cand, axis=1)
    kz_ref[...] = jnp.reshape(kz, (1, 1, kz.shape[0]))


def _tc_d1(xb_ref, iw_ref, h_ref, phi_ref,
           gnum_ref, waux_ref, saux_ref, pg_ref):
    pid = pl.program_id(0)

    @pl.when(pid == 0)
    def _():
        gnum_ref[...] = jnp.zeros_like(gnum_ref)
        waux_ref[...] = jnp.zeros_like(waux_ref)
        saux_ref[...] = jnp.zeros_like(saux_ref)
        pg_ref[...] = jnp.zeros_like(pg_ref)
    xb = xb_ref[0]
    iota_g = lax.broadcasted_iota(jnp.int32, (G, xb.shape[1]), 0)
    geb = jnp.where(xb == iota_g, 1.0, 0.0)
    gw = geb * iw_ref[0]
    ones = jnp.ones((xb.shape[1], NI), jnp.float32)
    gnum_ref[...] += jnp.dot(gw, h_ref[...], preferred_element_type=jnp.float32)
    waux_ref[...] += jnp.dot(gw, ones, preferred_element_type=jnp.float32)
    saux_ref[...] += jnp.dot(geb, ones, preferred_element_type=jnp.float32)
    pg_ref[...] += jnp.dot(geb, phi_ref[...],
                           preferred_element_type=jnp.float32)


def _tc_d2(gnum_ref, waux_ref, wmu_ref, bmu_ref, wlv_ref, blv_ref, eps_ref,
           theta_ref, kl1_ref):
    g = gnum_ref[...] / (waux_ref[:, 0:1] + 1e-10)
    mu = jnp.dot(g, wmu_ref[...], preferred_element_type=jnp.float32) + bmu_ref[...]
    lv = jnp.dot(g, wlv_ref[...], preferred_element_type=jnp.float32) + blv_ref[...]
    kl1 = 0.5 * jnp.sum(mu * mu + jnp.exp(lv) - lv - 1.0, axis=1, keepdims=True)
    kl1_ref[...] = jnp.concatenate([kl1, jnp.zeros((G, 7), jnp.float32)], axis=1)
    t = mu + eps_ref[...] * jnp.exp(0.5 * lv)
    iota_k = lax.broadcasted_iota(jnp.int32, t.shape, 1)
    t = jnp.where(iota_k < K, t, -1e30)
    tm = jnp.max(t, axis=1, keepdims=True)
    te = jnp.exp(t - tm)
    theta_ref[...] = te / jnp.sum(te, axis=1, keepdims=True)


def _tc_e(xb_ref, iw_ref, phi_ref, qn_ref, h0b_ref, tvp_ref, theta_ref,
          pg_ref, mzb_ref, sega_ref, segb_ref):
    pid = pl.program_id(0)

    @pl.when(pid == 0)
    def _():
        sega_ref[...] = jnp.zeros_like(sega_ref)
        segb_ref[...] = jnp.zeros_like(segb_ref)
    xb = xb_ref[0]
    iota_g = lax.broadcasted_iota(jnp.int32, (G, xb.shape[1]), 0)
    geb = jnp.where(xb == iota_g, 1.0, 0.0)
    phi = phi_ref[...]
    thx = lax.dot_general(geb, theta_ref[...], (((0,), (0,)), ((), ())),
                          preferred_element_type=jnp.float32)
    kl2n = jnp.sum(phi * jnp.log(phi / (thx + 1e-10) + 1e-10), axis=1)
    tlogit = lax.dot_general(h0b_ref[...], tvp_ref[...], (((1,), (1,)), ((), ())),
                             preferred_element_type=jnp.float32)
    logzb = mzb_ref[2:3, :]
    beta_s = jnp.exp(tlogit - logzb)
    rwn = jnp.sum(phi * jnp.log(beta_s + 1e-6), axis=1)
    pgath = lax.dot_general(geb, pg_ref[...], (((0,), (0,)), ((), ())),
                            preferred_element_type=jnp.float32)
    n1n = jnp.sum(qn_ref[...] * (pgath - phi), axis=1)
    nb = kl2n.shape[0]
    zeros6 = jnp.zeros((nb, 6), jnp.float32)
    s2 = jnp.concatenate([kl2n[:, None], rwn[:, None], zeros6], axis=1)
    s1 = jnp.concatenate([n1n[:, None], zeros6, jnp.zeros((nb, 1), jnp.float32)],
                         axis=1)
    gw = geb * iw_ref[0]
    sega_ref[...] += jnp.dot(gw, s2, preferred_element_type=jnp.float32)
    segb_ref[...] += jnp.dot(geb, s1, preferred_element_type=jnp.float32)


def _tc_f(wvb_ref, tvp_ref, t1_ref, t2_ref, at_ref, bt_ref, mzb_ref):
    pid = pl.program_id(0)

    @pl.when(pid == 0)
    def _():
        mzb_ref[...] = jnp.zeros_like(mzb_ref)
        mzb_ref[0:1, :] = jnp.full((1, KP), -1e30, jnp.float32)
    wvb = wvb_ref[...]
    at_ref[...] = lax.dot_general(wvb, t1_ref[...], (((1,), (1,)), ((), ())),
                                  preferred_element_type=jnp.float32)
    bt_ref[...] = lax.dot_general(wvb, t2_ref[...], (((1,), (1,)), ((), ())),
                                  preferred_element_type=jnp.float32)
    st = lax.dot_general(wvb, tvp_ref[...], (((1,), (1,)), ((), ())),
                         preferred_element_type=jnp.float32)
    r = lax.broadcasted_iota(jnp.int32, st.shape, 0) + pid * st.shape[0]
    st = jnp.where(r < V, st, -1e30)
    bm = jnp.max(st, axis=0, keepdims=True)
    m_old = mzb_ref[0:1, :]
    s_old = mzb_ref[1:2, :]
    m_new = jnp.maximum(m_old, bm)
    s_new = s_old * jnp.exp(m_old - m_new) + jnp.sum(jnp.exp(st - m_new),
                                                     axis=0, keepdims=True)
    mzb_ref[0:1, :] = m_new
    mzb_ref[1:2, :] = s_new

    @pl.when(pid == pl.num_programs(0) - 1)
    def _():
        mzb_ref[2:3, :] = m_new + jnp.log(s_new)


def _tc_g(a_ref, b_ref, mze_ref):
    pid = pl.program_id(0)

    @pl.when(pid == 0)
    def _():
        mze_ref[...] = jnp.zeros_like(mze_ref)
        mze_ref[0:1, :] = jnp.full((1, KP), -1e30, jnp.float32)
    s = a_ref[...] + b_ref[...]
    r = lax.broadcasted_iota(jnp.int32, s.shape, 0) + pid * s.shape[0]
    s = jnp.where(r < EW, s, -1e30)
    bm = jnp.max(s, axis=0, keepdims=True)
    m_old = mze_ref[0:1, :]
    s_old = mze_ref[1:2, :]
    m_new = jnp.maximum(m_old, bm)
    s_new = s_old * jnp.exp(m_old - m_new) + jnp.sum(jnp.exp(s - m_new),
                                                     axis=0, keepdims=True)
    mze_ref[0:1, :] = m_new
    mze_ref[1:2, :] = s_new

    @pl.when(pid == pl.num_programs(0) - 1)
    def _():
        mze_ref[2:3, :] = m_new + jnp.log(s_new)


def _tc_h(qpe0_ref, phi1_ref, qnui_ref, phiuj_ref, scal_ref,
          out1_ref, zew_ref, bes_ref):
    pid = pl.program_id(0)

    @pl.when(pid == 0)
    def _():
        out1_ref[...] = jnp.zeros_like(out1_ref)
        zew_ref[...] = jnp.zeros_like(zew_ref)
        bes_ref[...] = jnp.zeros_like(bes_ref)
    scal = scal_ref[...]
    nb = scal.shape[0]
    valid = scal[:, 0:1]
    pe = jnp.sum(qpe0_ref[...] * phi1_ref[...], axis=1, keepdims=True)
    nev = jnp.sum(qnui_ref[...] * phiuj_ref[...], axis=1, keepdims=True) * valid
    ones = jnp.ones((nb, 1), jnp.float32)
    zcol = jnp.zeros((nb, 1), jnp.float32)
    iota_g = lax.broadcasted_iota(jnp.int32, (nb, G), 1)
    iota_k = lax.broadcasted_iota(jnp.int32, (nb, KP), 1)
    gebt = jnp.where(scal[:, 2:3].astype(jnp.int32) == iota_g, 1.0, 0.0)
    ggit = jnp.where(scal[:, 1:2].astype(jnp.int32) == iota_g, 1.0, 0.0)
    k1h = jnp.where(scal[:, 3:4].astype(jnp.int32) == iota_k, 1.0, 0.0)
    zw = scal[:, 6:7] * scal[:, 4:5]
    bv = zw * jnp.exp(scal[:, 5:6])
    s6a = jnp.concatenate([pe, ones, zcol, zcol, zcol, zcol, zcol, zcol], axis=1)
    s6b = jnp.concatenate([zcol, zcol, nev, valid, zcol, zcol, zcol, zcol],
                          axis=1)
    out1_ref[...] += (
        lax.dot_general(gebt, s6a, (((0,), (0,)), ((), ())),
                        preferred_element_type=jnp.float32)
        + lax.dot_general(ggit, s6b, (((0,), (0,)), ((), ())),
                          preferred_element_type=jnp.float32))
    zew_ref[...] += lax.dot_general(gebt, k1h * zw, (((0,), (0,)), ((), ())),
                                    preferred_element_type=jnp.float32)
    bes_ref[...] += lax.dot_general(gebt, k1h * bv, (((0,), (0,)), ((), ())),
                                    preferred_element_type=jnp.float32)


def _tc_i(out1_ref, sega_ref, segb_ref, kl1_ref, saux_ref, zew_ref, bes_ref,
          mze_ref, out_ref):
    out1 = out1_ref[...]
    p_edge = out1[:, 0]
    np_ = out1[:, 1]
    necorr = out1[:, 2]
    cnt = out1[:, 3]
    kl2 = sega_ref[:, 0]
    recon_word = -sega_ref[:, 1]
    n1g = segb_ref[:, 0]
    kl1 = kl1_ref[:, 0]
    sizes = saux_ref[:, 0]
    n_edge1 = n1g - necorr
    nn = sizes * (sizes - 1.0) - cnt
    recon_structure = -(p_edge + n_edge1 / (nn + 1e-6) * np_)
    logze = mze_ref[2:3, :]
    bes = bes_ref[...] * jnp.exp(-logze)
    lr = (jnp.log(jnp.clip(bes, 1e-10, None))
          - jnp.log(jnp.clip(zew_ref[...], 1e-10, None)))
    recon_edge = -jnp.sum(lr, axis=1)
    loss = recon_edge + recon_word + kl1 + kl2 + recon_structure
    vals = [loss, recon_word, recon_edge, recon_structure, p_edge, kl1, kl2]
    r = lax.broadcasted_iota(jnp.int32, (8, NI), 0)
    c = lax.broadcasted_iota(jnp.int32, (8, NI), 1)
    acc = jnp.zeros((8, NI), jnp.float32)
    for i, v in enumerate(vals):
        mv = jnp.sum(v) / G
        acc = acc + jnp.where(jnp.logical_and(r == i, c == 0), mv, 0.0)
    out_ref[...] = acc


def _f32(x):
    return jnp.asarray(x, jnp.float32)


def kernel(idx_x, x_batch, idx_w, edge_w, edge_id, edge_id_batch, edge_index,
           whole_edge, word_vec, word_vec_beta, topic_vec, topic_edge_vec,
           W_topic, W_enc, b_enc, W_phi, b_phi, W_mu, b_mu, W_lv, b_lv):
    key = jax.random.key(42)
    idx_x = idx_x.astype(jnp.int32)
    x_batch = x_batch.astype(jnp.int32)
    edge_id = edge_id.astype(jnp.int32)
    edge_id_batch = edge_id_batch.astype(jnp.int32)
    e0 = edge_index[0].astype(jnp.int32)
    e1 = edge_index[1].astype(jnp.int32)
    we0 = whole_edge[0].astype(jnp.int32)
    we1 = whole_edge[1].astype(jnp.int32)
    word_vec = _f32(word_vec)
    wvb = _f32(word_vec_beta)

    skey = jnp.sort(e0 * N + e1)
    uniq = jnp.concatenate(
        [jnp.ones((1,), jnp.float32),
         (skey[1:] != skey[:-1]).astype(jnp.float32)])

    tvcat = jnp.zeros((KP, 3 * NI), jnp.float32).at[:K].set(
        jnp.concatenate([_f32(topic_vec), _f32(topic_edge_vec)], axis=-1))
    tvp = jnp.zeros((KP, NI), jnp.float32).at[:K].set(_f32(topic_vec))
    t1p = jnp.zeros((KP, NI), jnp.float32).at[:K].set(_f32(topic_edge_vec[:, :NI]))
    t2p = jnp.zeros((KP, NI), jnp.float32).at[:K].set(_f32(topic_edge_vec[:, NI:]))
    wphi_p = jnp.zeros((NI, KP), jnp.float32).at[:, :K].set(_f32(W_phi))
    bphi_p = jnp.full((1, KP), -1e30, jnp.float32).at[0, :K].set(_f32(b_phi))
    wmu_p = jnp.zeros((NI, KP), jnp.float32).at[:, :K].set(_f32(W_mu))
    bmu_p = jnp.zeros((1, KP), jnp.float32).at[0, :K].set(_f32(b_mu))
    wlv_p = jnp.zeros((NI, KP), jnp.float32).at[:, :K].set(_f32(W_lv))
    blv_p = jnp.zeros((1, KP), jnp.float32).at[0, :K].set(_f32(b_lv))
    benc = _f32(b_enc).reshape(1, NI)
    wvb_p = jnp.zeros((VP, NI), jnp.float32).at[:V].set(wvb)
    we0p = jnp.zeros((EWP,), jnp.int32).at[:EW].set(we0)
    we1p = jnp.zeros((EWP,), jnp.int32).at[:EW].set(we1)

    eps = jax.random.normal(jax.random.fold_in(key, 1), (G, K))
    eps_p = jnp.zeros((G, KP), jnp.float32).at[:, :K].set(eps)
    u = jax.random.uniform(jax.random.fold_in(key, 2), (1, N, K),
                           minval=1e-10, maxval=1.0)
    gumb = -jnp.log(-jnp.log(u))[0]
    gumb_p = jnp.full((N, KP), -1e30, jnp.float32).at[:, :K].set(gumb)

    sc1, sc2, sc3 = _sc_kernels()
    h0, h0b, h0e, ui, uj, gi, val0 = sc1(word_vec, wvb, idx_x, e0, skey,
                                         x_batch)
    valid = val0 * uniq

    lpw, lnw = pl.pallas_call(
        _tc_a,
        out_shape=[jax.ShapeDtypeStruct((KP, KP), jnp.float32)] * 2,
    )(tvcat, _f32(W_topic))

    EB = 256
    e1_3d = e1.reshape(E // EB, 1, EB)
    ew_3d = _f32(edge_w).reshape(E // EB, 1, EB)
    msg = pl.pallas_call(
        _tc_b,
        grid=(E // EB,),
        in_specs=[
            pl.BlockSpec((1, 1, EB), lambda i: (i, 0, 0)),
            pl.BlockSpec((EB, NI), lambda i: (i, 0)),
            pl.BlockSpec((1, 1, EB), lambda i: (i, 0, 0)),
        ],
        out_specs=pl.BlockSpec((N, NI), lambda i: (0, 0)),
        out_shape=jax.ShapeDtypeStruct((N, NI), jnp.float32),
    )(e1_3d, h0e, ew_3d)

    NB = 512
    full64 = pl.BlockSpec((KP, KP), lambda i: (0, 0))
    h, phi, qp, qn, kz3 = pl.pallas_call(
        _tc_c,
        grid=(N // NB,),
        in_specs=[
            pl.BlockSpec((NB, NI), lambda i: (i, 0)),
            pl.BlockSpec((NB, NI), lambda i: (i, 0)),
            pl.BlockSpec((NI, NI), lambda i: (0, 0)),
            pl.BlockSpec((1, NI), lambda i: (0, 0)),
            pl.BlockSpec((NI, KP), lambda i: (0, 0)),
            pl.BlockSpec((1, KP), lambda i: (0, 0)),
            full64, full64,
            pl.BlockSpec((NB, KP), lambda i: (i, 0)),
        ],
        out_specs=[
            pl.BlockSpec((NB, NI), lambda i: (i, 0)),
            pl.BlockSpec((NB, KP), lambda i: (i, 0)),
            pl.BlockSpec((NB, KP), lambda i: (i, 0)),
            pl.BlockSpec((NB, KP), lambda i: (i, 0)),
            pl.BlockSpec((1, 1, NB), lambda i: (i, 0, 0)),
        ],
        out_shape=[
            jax.ShapeDtypeStruct((N, NI), jnp.float32),
            jax.ShapeDtypeStruct((N, KP), jnp.float32),
            jax.ShapeDtypeStruct((N, KP), jnp.float32),
            jax.ShapeDtypeStruct((N, KP), jnp.float32),
            jax.ShapeDtypeStruct((N // NB, 1, NB), jnp.int32),
        ],
    )(h0, msg, _f32(W_enc), benc, wphi_p, bphi_p, lpw, lnw, gumb_p)
    kz = kz3.reshape(N)

    xb_3d = x_batch.reshape(N // NB, 1, NB)
    iw_3d = _f32(idx_w).reshape(N // NB, 1, NB)
    gnum, waux, saux, pg = pl.pallas_call(
        _tc_d1,
        grid=(N // NB,),
        in_specs=[
            pl.BlockSpec((1, 1, NB), lambda i: (i, 0, 0)),
            pl.BlockSpec((1, 1, NB), lambda i: (i, 0, 0)),
            pl.BlockSpec((NB, NI), lambda i: (i, 0)),
            pl.BlockSpec((NB, KP), lambda i: (i, 0)),
        ],
        out_specs=[
            pl.BlockSpec((G, NI), lambda i: (0, 0)),
            pl.BlockSpec((G, NI), lambda i: (0, 0)),
            pl.BlockSpec((G, NI), lambda i: (0, 0)),
            pl.BlockSpec((G, KP), lambda i: (0, 0)),
        ],
        out_shape=[
            jax.ShapeDtypeStruct((G, NI), jnp.float32),
            jax.ShapeDtypeStruct((G, NI), jnp.float32),
            jax.ShapeDtypeStruct((G, NI), jnp.float32),
            jax.ShapeDtypeStruct((G, KP), jnp.float32),
        ],
    )(xb_3d, iw_3d, h, phi)

    theta, kl1b = pl.pallas_call(
        _tc_d2,
        out_shape=[
            jax.ShapeDtypeStruct((G, KP), jnp.float32),
            jax.ShapeDtypeStruct((G, 8), jnp.float32),
        ],
    )(gnum, waux, wmu_p, bmu_p, wlv_p, blv_p, eps_p)

    VB = 512
    at, bt, mzb = pl.pallas_call(
        _tc_f,
        grid=(VP // VB,),
        in_specs=[
            pl.BlockSpec((VB, NI), lambda i: (i, 0)),
            pl.BlockSpec((KP, NI), lambda i: (0, 0)),
            pl.BlockSpec((KP, NI), lambda i: (0, 0)),
            pl.BlockSpec((KP, NI), lambda i: (0, 0)),
        ],
        out_specs=[
            pl.BlockSpec((VB, KP), lambda i: (i, 0)),
            pl.BlockSpec((VB, KP), lambda i: (i, 0)),
            pl.BlockSpec((8, KP), lambda i: (0, 0)),
        ],
        out_shape=[
            jax.ShapeDtypeStruct((VP, KP), jnp.float32),
            jax.ShapeDtypeStruct((VP, KP), jnp.float32),
            jax.ShapeDtypeStruct((8, KP), jnp.float32),
        ],
    )(wvb_p, tvp, t1p, t2p)

    sega, segb = pl.pallas_call(
        _tc_e,
        grid=(N // NB,),
        in_specs=[
            pl.BlockSpec((1, 1, NB), lambda i: (i, 0, 0)),
            pl.BlockSpec((1, 1, NB), lambda i: (i, 0, 0)),
            pl.BlockSpec((NB, KP), lambda i: (i, 0)),
            pl.BlockSpec((NB, KP), lambda i: (i, 0)),
            pl.BlockSpec((NB, NI), lambda i: (i, 0)),
            pl.BlockSpec((KP, NI), lambda i: (0, 0)),
            pl.BlockSpec((G, KP), lambda i: (0, 0)),
            pl.BlockSpec((G, KP), lambda i: (0, 0)),
            pl.BlockSpec((8, KP), lambda i: (0, 0)),
        ],
        out_specs=[
            pl.BlockSpec((G, 8), lambda i: (0, 0)),
            pl.BlockSpec((G, 8), lambda i: (0, 0)),
        ],
        out_shape=[
            jax.ShapeDtypeStruct((G, 8), jnp.float32),
            jax.ShapeDtypeStruct((G, 8), jnp.float32),
        ],
    )(xb_3d, iw_3d, phi, qn, h0b, tvp, theta, pg, mzb)

    phi1, qpe0, qnui, phiuj, a100, b100 = sc2(
        phi, qp, qn, at, bt, e0, e1, ui, uj, we0p, we1p)

    WB = 512
    mze = pl.pallas_call(
        _tc_g,
        grid=(EWP // WB,),
        in_specs=[
            pl.BlockSpec((WB, KP), lambda i: (i, 0)),
            pl.BlockSpec((WB, KP), lambda i: (i, 0)),
        ],
        out_specs=pl.BlockSpec((8, KP), lambda i: (0, 0)),
        out_shape=jax.ShapeDtypeStruct((8, KP), jnp.float32),
    )(a100, b100)

    sval, kz0, same = sc3(a100, b100, edge_id, kz, e0, e1)

    scal = jnp.stack(
        [valid, gi.astype(jnp.float32), edge_id_batch.astype(jnp.float32),
         kz0.astype(jnp.float32), same, sval, _f32(edge_w),
         jnp.zeros((E,), jnp.float32)], axis=1)

    EB2 = 512
    out1, zew, bes = pl.pallas_call(
        _tc_h,
        grid=(E // EB2,),
        in_specs=[
            pl.BlockSpec((EB2, KP), lambda i: (i, 0)),
            pl.BlockSpec((EB2, KP), lambda i: (i, 0)),
            pl.BlockSpec((EB2, KP), lambda i: (i, 0)),
            pl.BlockSpec((EB2, KP), lambda i: (i, 0)),
            pl.BlockSpec((EB2, 8), lambda i: (i, 0)),
        ],
        out_specs=[
            pl.BlockSpec((G, 8), lambda i: (0, 0)),
            pl.BlockSpec((G, KP), lambda i: (0, 0)),
            pl.BlockSpec((G, KP), lambda i: (0, 0)),
        ],
        out_shape=[
            jax.ShapeDtypeStruct((G, 8), jnp.float32),
            jax.ShapeDtypeStruct((G, KP), jnp.float32),
            jax.ShapeDtypeStruct((G, KP), jnp.float32),
        ],
    )(qpe0, phi1, qnui, phiuj, scal)

    out = pl.pallas_call(
        _tc_i,
        out_shape=jax.ShapeDtypeStruct((8, NI), jnp.float32),
    )(out1, sega, segb, kl1b, saux, zew, bes, mze)

    return (out[0, 0], out[1, 0], out[2, 0], out[3, 0], out[4, 0],
            out[5, 0], out[6, 0])


# speedup vs baseline: 3.6385x; 1.2546x over previous
import functools
import jax
import jax.numpy as jnp
from jax import lax
from jax.experimental import pallas as pl
from jax.experimental.pallas import tpu as pltpu
from jax.experimental.pallas import tpu_sc as plsc

N = 4096
E = 65536
V = 50000
NI = 128
K = 50
KP = 128
G = 64
EW = 100000
VP = 50176
EWP = 100352
TEMP = 0.5

NC = 2
NS = 16
NW = NC * NS
EPW = E // NW
NPW = N // NW
WPW = EWP // NW

def _wid():
    return lax.axis_index("s") * NC + lax.axis_index("c")


_SC1_TYPES = dict(
    out_type=[
        jax.ShapeDtypeStruct((N, NI), jnp.float32),
        jax.ShapeDtypeStruct((N, NI), jnp.float32),
        jax.ShapeDtypeStruct((NC, N, NI), jnp.float32),
        jax.ShapeDtypeStruct((E,), jnp.int32),
        jax.ShapeDtypeStruct((E,), jnp.int32),
        jax.ShapeDtypeStruct((E,), jnp.int32),
        jax.ShapeDtypeStruct((E,), jnp.float32),
    ],
    scratch_types=[
        pltpu.VMEM((N,), jnp.int32),
        pltpu.VMEM((N,), jnp.int32),
        pltpu.VMEM((NPW,), jnp.int32),
        pltpu.VMEM((512, NI), jnp.float32),
        pltpu.VMEM((EPW,), jnp.int32),
        pltpu.VMEM((512,), jnp.int32),
        pltpu.VMEM((512,), jnp.int32),
        pltpu.VMEM((512,), jnp.int32),
        pltpu.VMEM((512,), jnp.int32),
        pltpu.VMEM((EPW,), jnp.float32),
        pltpu.VMEM((EPW,), jnp.int32),
        pltpu.VMEM((EPW,), jnp.int32),
        pltpu.VMEM((EPW,), jnp.int32),
        pltpu.VMEM((EPW,), jnp.int32),
        pltpu.VMEM((EPW,), jnp.int32),
        pltpu.VMEM((EPW,), jnp.float32),
        pltpu.VMEM_SHARED((N, NI), jnp.float32),
        pltpu.SemaphoreType.DMA,
    ],
)


def _sc1(wv_h, wvb_h, idxx_h, e0_h, e1_h, ew_h, zinit_h, skey_h, xb_h,
         h0_o, h0b_o, msg_o, ui_o, uj_o, gi_o, val_o,
         idxx_v, xb_v, idxn_v, rowbuf, e0_v, e1c0, e1c1, e1c2, e1c3,
         ew_v, idx2_v, sk_v,
         ui_v, uj_v, gi_v, val_v, acc_sh, sem):
    cid = lax.axis_index("c")
    sid = lax.axis_index("s")
    wid = _wid()
    base_n = wid * NPW
    base_e = wid * EPW
    @pl.when(sid == 0)
    def _():
        pltpu.sync_copy(zinit_h, acc_sh)
    plsc.subcore_barrier()
    pltpu.sync_copy(idxx_h.at[pl.ds(base_n, NPW)], idxn_v)
    pltpu.async_copy(wv_h.at[idxn_v], rowbuf.at[pl.ds(0, NPW)], sem).wait()
    pltpu.sync_copy(rowbuf.at[pl.ds(0, NPW)], h0_o.at[pl.ds(base_n, NPW)])
    pltpu.async_copy(wvb_h.at[idxn_v], rowbuf.at[pl.ds(0, NPW)], sem).wait()
    pltpu.sync_copy(rowbuf.at[pl.ds(0, NPW)], h0b_o.at[pl.ds(base_n, NPW)])
    pltpu.sync_copy(idxx_h, idxx_v)
    pltpu.sync_copy(xb_h, xb_v)
    pltpu.sync_copy(e0_h.at[pl.ds(base_e, EPW)], e0_v)

    def body_i2(i, _):
        ev = e0_v[pl.ds(i * 16, 16)]
        idx2_v[pl.ds(i * 16, 16)] = plsc.load_gather(idxx_v, [ev])
        return 0
    lax.fori_loop(0, EPW // 16, body_i2, 0)
    pltpu.sync_copy(ew_h.at[pl.ds(base_e, EPW)], ew_v)
    e1bufs = [e1c0, e1c1, e1c2, e1c3]
    for c in range(EPW // 512):
        pltpu.sync_copy(e1_h.at[pl.ds(base_e + c * 512, 512)], e1bufs[c])
        pltpu.async_copy(wv_h.at[idx2_v.at[pl.ds(c * 512, 512)]], rowbuf,
                         sem).wait()

        def body_w(j, _):
            wv16 = plsc.load_gather(ew_v, [jnp.full((16,), c * 512, jnp.int32)
                                           + j])
            for g in range(NI // 16):
                sl = pl.ds(g * 16, 16)
                rowbuf[j, sl] = rowbuf[j, sl] * wv16
            return 0
        lax.fori_loop(0, 512, body_w, 0)
        pltpu.sync_copy(rowbuf, acc_sh.at[e1bufs[c]], add=True)
    plsc.subcore_barrier()
    pltpu.sync_copy(acc_sh.at[pl.ds(sid * (N // NS), N // NS)],
                    msg_o.at[cid, pl.ds(sid * (N // NS), N // NS)])
    pltpu.sync_copy(skey_h.at[pl.ds(base_e, EPW)], sk_v)

    def body_uk(i, _):
        sl = pl.ds(i * 16, 16)
        kv = sk_v[sl]
        uiv = lax.shift_right_logical(kv, 12)
        ujv = lax.bitwise_and(kv, 4095)
        giv = plsc.load_gather(xb_v, [uiv])
        gjv = plsc.load_gather(xb_v, [ujv])
        ui_v[sl] = uiv
        uj_v[sl] = ujv
        gi_v[sl] = giv
        ok = jnp.logical_and(giv == gjv, uiv != ujv)
        val_v[sl] = jnp.where(ok, 1.0, 0.0).astype(jnp.float32)
        return 0
    lax.fori_loop(0, EPW // 16, body_uk, 0)
    pltpu.sync_copy(ui_v, ui_o.at[pl.ds(base_e, EPW)])
    pltpu.sync_copy(uj_v, uj_o.at[pl.ds(base_e, EPW)])
    pltpu.sync_copy(gi_v, gi_o.at[pl.ds(base_e, EPW)])
    pltpu.sync_copy(val_v, val_o.at[pl.ds(base_e, EPW)])


_SC2_TYPES = dict(
    out_type=[
        jax.ShapeDtypeStruct((E, KP), jnp.float32),
        jax.ShapeDtypeStruct((E, KP), jnp.float32),
        jax.ShapeDtypeStruct((E, KP), jnp.float32),
        jax.ShapeDtypeStruct((E, KP), jnp.float32),
        jax.ShapeDtypeStruct((EWP, KP), jnp.float32),
        jax.ShapeDtypeStruct((EWP, KP), jnp.float32),
    ],
    scratch_types=[
        pltpu.VMEM((EPW,), jnp.int32),
        pltpu.VMEM((WPW,), jnp.int32),
        pltpu.VMEM((512, KP), jnp.float32),
        pltpu.SemaphoreType.DMA,
    ],
)


def _sc2(phi_h, qp_h, qn_h, at_h, bt_h, e0_h, e1_h, ui_h, uj_h,
         we0_h, we1_h,
         phi1_o, qpe0_o, qnui_o, phiuj_o, a100_o, b100_o,
         idx_v, bigidx_v, rowbuf, sem):
    wid = _wid()
    base_e = wid * EPW
    base_w = wid * WPW
    for idx_h, tab_h, out_o in ((e1_h, phi_h, phi1_o), (e0_h, qp_h, qpe0_o),
                                (ui_h, qn_h, qnui_o), (uj_h, phi_h, phiuj_o)):
        pltpu.sync_copy(idx_h.at[pl.ds(base_e, EPW)], idx_v)
        for c in range(EPW // 512):
            pltpu.async_copy(tab_h.at[idx_v.at[pl.ds(c * 512, 512)]], rowbuf,
                             sem).wait()
            pltpu.sync_copy(rowbuf, out_o.at[pl.ds(base_e + c * 512, 512)])
    for widx_h, tab_h, out_o in ((we0_h, at_h, a100_o), (we1_h, bt_h, b100_o)):
        pltpu.sync_copy(widx_h.at[pl.ds(base_w, WPW)], bigidx_v)
        for c in range(WPW // 448):
            pltpu.async_copy(tab_h.at[bigidx_v.at[pl.ds(c * 448, 448)]],
                             rowbuf.at[pl.ds(0, 448)], sem).wait()
            pltpu.sync_copy(rowbuf.at[pl.ds(0, 448)],
                            out_o.at[pl.ds(base_w + c * 448, 448)])


_SC3_TYPES = dict(
    out_type=[
        jax.ShapeDtypeStruct((E,), jnp.float32),
        jax.ShapeDtypeStruct((E,), jnp.int32),
        jax.ShapeDtypeStruct((E,), jnp.float32),
    ],
    scratch_types=[
        pltpu.VMEM((N,), jnp.int32),
        pltpu.VMEM((EPW,), jnp.int32),
        pltpu.VMEM((EPW,), jnp.int32),
        pltpu.VMEM((EPW,), jnp.int32),
        pltpu.VMEM((EPW,), jnp.int32),
        pltpu.VMEM((EPW,), jnp.float32),
        pltpu.VMEM((EPW,), jnp.float32),
        pltpu.VMEM((512, KP), jnp.float32),
        pltpu.SemaphoreType.DMA,
    ],
)


def _sc3(a100_h, b100_h, eid_h, kz_h, e0_h, e1_h,
         sval_o, kz0_o, same_o,
         kz_v, eid_v, e0_v, e1_v, kz0_v, sv_v, same_v, rowbuf, sem):
    wid = _wid()
    base_e = wid * EPW
    pltpu.sync_copy(kz_h, kz_v)
    pltpu.sync_copy(eid_h.at[pl.ds(base_e, EPW)], eid_v)
    pltpu.sync_copy(e0_h.at[pl.ds(base_e, EPW)], e0_v)
    pltpu.sync_copy(e1_h.at[pl.ds(base_e, EPW)], e1_v)

    def body_kz(i, _):
        sl = pl.ds(i * 16, 16)
        k0 = plsc.load_gather(kz_v, [e0_v[sl]])
        k1 = plsc.load_gather(kz_v, [e1_v[sl]])
        kz0_v[sl] = k0
        same_v[sl] = jnp.where(k0 == k1, 1.0, 0.0).astype(jnp.float32)
        return 0
    lax.fori_loop(0, EPW // 16, body_kz, 0)

    rows16 = lax.iota(jnp.int32, 16)
    for c in range(EPW // 512):
        pltpu.async_copy(a100_h.at[eid_v.at[pl.ds(c * 512, 512)]], rowbuf,
                         sem).wait()

        def body_pa(j, _):
            sl = pl.ds(c * 512 + j * 16, 16)
            va = plsc.load_gather(rowbuf, [rows16 + j * 16, kz0_v[sl]])
            sv_v[sl] = va
            return 0
        lax.fori_loop(0, 512 // 16, body_pa, 0)
        pltpu.async_copy(b100_h.at[eid_v.at[pl.ds(c * 512, 512)]], rowbuf,
                         sem).wait()

        def body_pb(j, _):
            sl = pl.ds(c * 512 + j * 16, 16)
            vb = plsc.load_gather(rowbuf, [rows16 + j * 16, kz0_v[sl]])
            sv_v[sl] = sv_v[sl] + vb
            return 0
        lax.fori_loop(0, 512 // 16, body_pb, 0)
    pltpu.sync_copy(sv_v, sval_o.at[pl.ds(base_e, EPW)])
    pltpu.sync_copy(kz0_v, kz0_o.at[pl.ds(base_e, EPW)])
    pltpu.sync_copy(same_v, same_o.at[pl.ds(base_e, EPW)])


@functools.lru_cache(maxsize=1)
def _sc_kernels():
    mesh = plsc.VectorSubcoreMesh(core_axis_name="c", subcore_axis_name="s")
    cp = pltpu.CompilerParams(needs_layout_passes=False)
    sc1 = pl.kernel(_sc1, mesh=mesh, compiler_params=cp, **_SC1_TYPES)
    sc2 = pl.kernel(_sc2, mesh=mesh, compiler_params=cp, **_SC2_TYPES)
    sc3 = pl.kernel(_sc3, mesh=mesh, compiler_params=cp, **_SC3_TYPES)
    return sc1, sc2, sc3


def _tc_a(tvcat_ref, wt_ref, lpw_ref, lnw_ref):
    tv = jnp.dot(tvcat_ref[...], wt_ref[...].T,
                 preferred_element_type=jnp.float32)
    s = jnp.dot(tv, tv.T, preferred_element_type=jnp.float32)
    wm = jnp.clip(jax.nn.sigmoid(s), 1e-6, 1.0 - 1e-6)
    r = lax.broadcasted_iota(jnp.int32, (KP, KP), 0)
    c = lax.broadcasted_iota(jnp.int32, (KP, KP), 1)
    mask = jnp.where(jnp.logical_and(r < K, c < K), 1.0, 0.0)
    lpw_ref[...] = jnp.log(wm) * mask
    lnw_ref[...] = jnp.log(1.0 - wm) * mask


def _tc_c(h0_ref, msga_ref, msgb_ref, wenc_ref, benc_ref, wphi_ref, bphi_ref,
          lpw_ref, lnw_ref, gumb_ref,
          h_ref, phi_ref, qp_ref, qn_ref, kz_ref):
    x = h0_ref[...] + msga_ref[...] + msgb_ref[...]
    h = jax.nn.relu(jnp.dot(x, wenc_ref[...],
                            preferred_element_type=jnp.float32) + benc_ref[...])
    h_ref[...] = h
    logits = jnp.dot(h, wphi_ref[...],
                     preferred_element_type=jnp.float32) + bphi_ref[...]
    m = jnp.max(logits, axis=1, keepdims=True)
    ex = jnp.exp(logits - m)
    phi = ex / jnp.sum(ex, axis=1, keepdims=True)
    phi_ref[...] = phi
    qp_ref[...] = jnp.dot(phi, lpw_ref[...], preferred_element_type=jnp.float32)
    qn_ref[...] = jnp.dot(phi, lnw_ref[...], preferred_element_type=jnp.float32)
    gl = jnp.log(phi + 1e-20) + gumb_ref[...]
    gm = jnp.max(gl, axis=1, keepdims=True)
    iota_k = lax.broadcasted_iota(jnp.int32, gl.shape, 1)
    cand = jnp.where(gl >= gm, iota_k, jnp.int32(10**9))
    kz = jnp.min(cand, axis=1)
    kz_ref[...] = jnp.reshape(kz, (1, 1, kz.shape[0]))


def _tc_d1(xb_ref, iw_ref, h_ref, phi_ref,
           gnum_ref, waux_ref, saux_ref, pg_ref):
    pid = pl.program_id(0)

    @pl.when(pid == 0)
    def _():
        gnum_ref[...] = jnp.zeros_like(gnum_ref)
        waux_ref[...] = jnp.zeros_like(waux_ref)
        saux_ref[...] = jnp.zeros_like(saux_ref)
        pg_ref[...] = jnp.zeros_like(pg_ref)
    xb = xb_ref[0]
    iota_g = lax.broadcasted_iota(jnp.int32, (G, xb.shape[1]), 0)
    geb = jnp.where(xb == iota_g, 1.0, 0.0)
    gw = geb * iw_ref[0]
    ones = jnp.ones((xb.shape[1], NI), jnp.float32)
    gnum_ref[...] += jnp.dot(gw, h_ref[...], preferred_element_type=jnp.float32)
    waux_ref[...] += jnp.dot(gw, ones, preferred_element_type=jnp.float32)
    saux_ref[...] += jnp.dot(geb, ones, preferred_element_type=jnp.float32)
    pg_ref[...] += jnp.dot(geb, phi_ref[...],
                           preferred_element_type=jnp.float32)


def _tc_d2(gnum_ref, waux_ref, wmu_ref, bmu_ref, wlv_ref, blv_ref, eps_ref,
           theta_ref, kl1_ref):
    g = gnum_ref[...] / (waux_ref[:, 0:1] + 1e-10)
    mu = jnp.dot(g, wmu_ref[...], preferred_element_type=jnp.float32) + bmu_ref[...]
    lv = jnp.dot(g, wlv_ref[...], preferred_element_type=jnp.float32) + blv_ref[...]
    kl1 = 0.5 * jnp.sum(mu * mu + jnp.exp(lv) - lv - 1.0, axis=1, keepdims=True)
    kl1_ref[...] = jnp.concatenate([kl1, jnp.zeros((G, 7), jnp.float32)], axis=1)
    t = mu + eps_ref[...] * jnp.exp(0.5 * lv)
    iota_k = lax.broadcasted_iota(jnp.int32, t.shape, 1)
    t = jnp.where(iota_k < K, t, -1e30)
    tm = jnp.max(t, axis=1, keepdims=True)
    te = jnp.exp(t - tm)
    theta_ref[...] = te / jnp.sum(te, axis=1, keepdims=True)


def _tc_e(xb_ref, iw_ref, phi_ref, qn_ref, h0b_ref, tvp_ref, theta_ref,
          pg_ref, mzb_ref, sega_ref, segb_ref):
    pid = pl.program_id(0)

    @pl.when(pid == 0)
    def _():
        sega_ref[...] = jnp.zeros_like(sega_ref)
        segb_ref[...] = jnp.zeros_like(segb_ref)
    xb = xb_ref[0]
    iota_g = lax.broadcasted_iota(jnp.int32, (G, xb.shape[1]), 0)
    geb = jnp.where(xb == iota_g, 1.0, 0.0)
    phi = phi_ref[...]
    thx = lax.dot_general(geb, theta_ref[...], (((0,), (0,)), ((), ())),
                          preferred_element_type=jnp.float32)
    kl2n = jnp.sum(phi * jnp.log(phi / (thx + 1e-10) + 1e-10), axis=1)
    tlogit = lax.dot_general(h0b_ref[...], tvp_ref[...], (((1,), (1,)), ((), ())),
                             preferred_element_type=jnp.float32)
    logzb = mzb_ref[2:3, :]
    beta_s = jnp.exp(tlogit - logzb)
    rwn = jnp.sum(phi * jnp.log(beta_s + 1e-6), axis=1)
    pgath = lax.dot_general(geb, pg_ref[...], (((0,), (0,)), ((), ())),
                            preferred_element_type=jnp.float32)
    n1n = jnp.sum(qn_ref[...] * (pgath - phi), axis=1)
    nb = kl2n.shape[0]
    zeros6 = jnp.zeros((nb, 6), jnp.float32)
    s2 = jnp.concatenate([kl2n[:, None], rwn[:, None], zeros6], axis=1)
    s1 = jnp.concatenate([n1n[:, None], zeros6, jnp.zeros((nb, 1), jnp.float32)],
                         axis=1)
    gw = geb * iw_ref[0]
    sega_ref[...] += jnp.dot(gw, s2, preferred_element_type=jnp.float32)
    segb_ref[...] += jnp.dot(geb, s1, preferred_element_type=jnp.float32)


def _tc_f(wvb_ref, tvp_ref, t1_ref, t2_ref, at_ref, bt_ref, mzb_ref):
    pid = pl.program_id(0)

    @pl.when(pid == 0)
    def _():
        mzb_ref[...] = jnp.zeros_like(mzb_ref)
        mzb_ref[0:1, :] = jnp.full((1, KP), -1e30, jnp.float32)
    wvb = wvb_ref[...]
    at_ref[...] = lax.dot_general(wvb, t1_ref[...], (((1,), (1,)), ((), ())),
                                  preferred_element_type=jnp.float32)
    bt_ref[...] = lax.dot_general(wvb, t2_ref[...], (((1,), (1,)), ((), ())),
                                  preferred_element_type=jnp.float32)
    st = lax.dot_general(wvb, tvp_ref[...], (((1,), (1,)), ((), ())),
                         preferred_element_type=jnp.float32)
    r = lax.broadcasted_iota(jnp.int32, st.shape, 0) + pid * st.shape[0]
    st = jnp.where(r < V, st, -1e30)
    bm = jnp.max(st, axis=0, keepdims=True)
    m_old = mzb_ref[0:1, :]
    s_old = mzb_ref[1:2, :]
    m_new = jnp.maximum(m_old, bm)
    s_new = s_old * jnp.exp(m_old - m_new) + jnp.sum(jnp.exp(st - m_new),
                                                     axis=0, keepdims=True)
    mzb_ref[0:1, :] = m_new
    mzb_ref[1:2, :] = s_new

    @pl.when(pid == pl.num_programs(0) - 1)
    def _():
        mzb_ref[2:3, :] = m_new + jnp.log(s_new)


def _tc_g(a_ref, b_ref, mze_ref):
    pid = pl.program_id(0)

    @pl.when(pid == 0)
    def _():
        mze_ref[...] = jnp.zeros_like(mze_ref)
        mze_ref[0:1, :] = jnp.full((1, KP), -1e30, jnp.float32)
    s = a_ref[...] + b_ref[...]
    r = lax.broadcasted_iota(jnp.int32, s.shape, 0) + pid * s.shape[0]
    s = jnp.where(r < EW, s, -1e30)
    bm = jnp.max(s, axis=0, keepdims=True)
    m_old = mze_ref[0:1, :]
    s_old = mze_ref[1:2, :]
    m_new = jnp.maximum(m_old, bm)
    s_new = s_old * jnp.exp(m_old - m_new) + jnp.sum(jnp.exp(s - m_new),
                                                     axis=0, keepdims=True)
    mze_ref[0:1, :] = m_new
    mze_ref[1:2, :] = s_new

    @pl.when(pid == pl.num_programs(0) - 1)
    def _():
        mze_ref[2:3, :] = m_new + jnp.log(s_new)


def _tc_h(qpe0_ref, phi1_ref, qnui_ref, phiuj_ref, scal_ref,
          out1_ref, zew_ref, bes_ref):
    pid = pl.program_id(0)

    @pl.when(pid == 0)
    def _():
        out1_ref[...] = jnp.zeros_like(out1_ref)
        zew_ref[...] = jnp.zeros_like(zew_ref)
        bes_ref[...] = jnp.zeros_like(bes_ref)
    scal = scal_ref[...]
    nb = scal.shape[0]
    valid = scal[:, 0:1]
    pe = jnp.sum(qpe0_ref[...] * phi1_ref[...], axis=1, keepdims=True)
    nev = jnp.sum(qnui_ref[...] * phiuj_ref[...], axis=1, keepdims=True) * valid
    ones = jnp.ones((nb, 1), jnp.float32)
    zcol = jnp.zeros((nb, 1), jnp.float32)
    iota_g = lax.broadcasted_iota(jnp.int32, (nb, G), 1)
    iota_k = lax.broadcasted_iota(jnp.int32, (nb, KP), 1)
    gebt = jnp.where(scal[:, 2:3].astype(jnp.int32) == iota_g, 1.0, 0.0)
    ggit = jnp.where(scal[:, 1:2].astype(jnp.int32) == iota_g, 1.0, 0.0)
    k1h = jnp.where(scal[:, 3:4].astype(jnp.int32) == iota_k, 1.0, 0.0)
    zw = scal[:, 6:7] * scal[:, 4:5]
    bv = zw * jnp.exp(scal[:, 5:6])
    s6a = jnp.concatenate([pe, ones, zcol, zcol, zcol, zcol, zcol, zcol], axis=1)
    s6b = jnp.concatenate([zcol, zcol, nev, valid, zcol, zcol, zcol, zcol],
                          axis=1)
    out1_ref[...] += (
        lax.dot_general(gebt, s6a, (((0,), (0,)), ((), ())),
                        preferred_element_type=jnp.float32)
        + lax.dot_general(ggit, s6b, (((0,), (0,)), ((), ())),
                          preferred_element_type=jnp.float32))
    zew_ref[...] += lax.dot_general(gebt, k1h * zw, (((0,), (0,)), ((), ())),
                                    preferred_element_type=jnp.float32)
    bes_ref[...] += lax.dot_general(gebt, k1h * bv, (((0,), (0,)), ((), ())),
                                    preferred_element_type=jnp.float32)


def _tc_i(out1_ref, sega_ref, segb_ref, kl1_ref, saux_ref, zew_ref, bes_ref,
          mze_ref, out_ref):
    out1 = out1_ref[...]
    p_edge = out1[:, 0]
    np_ = out1[:, 1]
    necorr = out1[:, 2]
    cnt = out1[:, 3]
    kl2 = sega_ref[:, 0]
    recon_word = -sega_ref[:, 1]
    n1g = segb_ref[:, 0]
    kl1 = kl1_ref[:, 0]
    sizes = saux_ref[:, 0]
    n_edge1 = n1g - necorr
    nn = sizes * (sizes - 1.0) - cnt
    recon_structure = -(p_edge + n_edge1 / (nn + 1e-6) * np_)
    logze = mze_ref[2:3, :]
    bes = bes_ref[...] * jnp.exp(-logze)
    lr = (jnp.log(jnp.clip(bes, 1e-10, None))
          - jnp.log(jnp.clip(zew_ref[...], 1e-10, None)))
    recon_edge = -jnp.sum(lr, axis=1)
    loss = recon_edge + recon_word + kl1 + kl2 + recon_structure
    vals = [loss, recon_word, recon_edge, recon_structure, p_edge, kl1, kl2]
    r = lax.broadcasted_iota(jnp.int32, (8, NI), 0)
    c = lax.broadcasted_iota(jnp.int32, (8, NI), 1)
    acc = jnp.zeros((8, NI), jnp.float32)
    for i, v in enumerate(vals):
        mv = jnp.sum(v) / G
        acc = acc + jnp.where(jnp.logical_and(r == i, c == 0), mv, 0.0)
    out_ref[...] = acc


def _f32(x):
    return jnp.asarray(x, jnp.float32)


def kernel(idx_x, x_batch, idx_w, edge_w, edge_id, edge_id_batch, edge_index,
           whole_edge, word_vec, word_vec_beta, topic_vec, topic_edge_vec,
           W_topic, W_enc, b_enc, W_phi, b_phi, W_mu, b_mu, W_lv, b_lv):
    key = jax.random.key(42)
    idx_x = idx_x.astype(jnp.int32)
    x_batch = x_batch.astype(jnp.int32)
    edge_id = edge_id.astype(jnp.int32)
    edge_id_batch = edge_id_batch.astype(jnp.int32)
    e0 = edge_index[0].astype(jnp.int32)
    e1 = edge_index[1].astype(jnp.int32)
    we0 = whole_edge[0].astype(jnp.int32)
    we1 = whole_edge[1].astype(jnp.int32)
    word_vec = _f32(word_vec)
    wvb = _f32(word_vec_beta)

    skey = jnp.sort(e0 * N + e1)
    uniq = jnp.concatenate(
        [jnp.ones((1,), jnp.float32),
         (skey[1:] != skey[:-1]).astype(jnp.float32)])

    tvcat = jnp.zeros((KP, 3 * NI), jnp.float32).at[:K].set(
        jnp.concatenate([_f32(topic_vec), _f32(topic_edge_vec)], axis=-1))
    tvp = jnp.zeros((KP, NI), jnp.float32).at[:K].set(_f32(topic_vec))
    t1p = jnp.zeros((KP, NI), jnp.float32).at[:K].set(_f32(topic_edge_vec[:, :NI]))
    t2p = jnp.zeros((KP, NI), jnp.float32).at[:K].set(_f32(topic_edge_vec[:, NI:]))
    wphi_p = jnp.zeros((NI, KP), jnp.float32).at[:, :K].set(_f32(W_phi))
    bphi_p = jnp.full((1, KP), -1e30, jnp.float32).at[0, :K].set(_f32(b_phi))
    wmu_p = jnp.zeros((NI, KP), jnp.float32).at[:, :K].set(_f32(W_mu))
    bmu_p = jnp.zeros((1, KP), jnp.float32).at[0, :K].set(_f32(b_mu))
    wlv_p = jnp.zeros((NI, KP), jnp.float32).at[:, :K].set(_f32(W_lv))
    blv_p = jnp.zeros((1, KP), jnp.float32).at[0, :K].set(_f32(b_lv))
    benc = _f32(b_enc).reshape(1, NI)
    wvb_p = jnp.zeros((VP, NI), jnp.float32).at[:V].set(wvb)
    we0p = jnp.zeros((EWP,), jnp.int32).at[:EW].set(we0)
    we1p = jnp.zeros((EWP,), jnp.int32).at[:EW].set(we1)

    eps = jax.random.normal(jax.random.fold_in(key, 1), (G, K))
    eps_p = jnp.zeros((G, KP), jnp.float32).at[:, :K].set(eps)
    u = jax.random.uniform(jax.random.fold_in(key, 2), (1, N, K),
                           minval=1e-10, maxval=1.0)
    gumb = -jnp.log(-jnp.log(u))[0]
    gumb_p = jnp.full((N, KP), -1e30, jnp.float32).at[:, :K].set(gumb)

    sc1, sc2, sc3 = _sc_kernels()
    zinit = jnp.zeros((N, NI), jnp.float32)
    h0, h0b, msg2, ui, uj, gi, val0 = sc1(word_vec, wvb, idx_x, e0, e1,
                                          _f32(edge_w), zinit, skey, x_batch)
    valid = val0 * uniq

    lpw, lnw = pl.pallas_call(
        _tc_a,
        out_shape=[jax.ShapeDtypeStruct((KP, KP), jnp.float32)] * 2,
    )(tvcat, _f32(W_topic))

    NB = 512
    full64 = pl.BlockSpec((KP, KP), lambda i: (0, 0))
    h, phi, qp, qn, kz3 = pl.pallas_call(
        _tc_c,
        grid=(N // NB,),
        in_specs=[
            pl.BlockSpec((NB, NI), lambda i: (i, 0)),
            pl.BlockSpec((NB, NI), lambda i: (i, 0)),
            pl.BlockSpec((NB, NI), lambda i: (i, 0)),
            pl.BlockSpec((NI, NI), lambda i: (0, 0)),
            pl.BlockSpec((1, NI), lambda i: (0, 0)),
            pl.BlockSpec((NI, KP), lambda i: (0, 0)),
            pl.BlockSpec((1, KP), lambda i: (0, 0)),
            full64, full64,
            pl.BlockSpec((NB, KP), lambda i: (i, 0)),
        ],
        out_specs=[
            pl.BlockSpec((NB, NI), lambda i: (i, 0)),
            pl.BlockSpec((NB, KP), lambda i: (i, 0)),
            pl.BlockSpec((NB, KP), lambda i: (i, 0)),
            pl.BlockSpec((NB, KP), lambda i: (i, 0)),
            pl.BlockSpec((1, 1, NB), lambda i: (i, 0, 0)),
        ],
        out_shape=[
            jax.ShapeDtypeStruct((N, NI), jnp.float32),
            jax.ShapeDtypeStruct((N, KP), jnp.float32),
            jax.ShapeDtypeStruct((N, KP), jnp.float32),
            jax.ShapeDtypeStruct((N, KP), jnp.float32),
            jax.ShapeDtypeStruct((N // NB, 1, NB), jnp.int32),
        ],
    )(h0, msg2[0], msg2[1], _f32(W_enc), benc, wphi_p, bphi_p, lpw, lnw,
      gumb_p)
    kz = kz3.reshape(N)

    xb_3d = x_batch.reshape(N // NB, 1, NB)
    iw_3d = _f32(idx_w).reshape(N // NB, 1, NB)
    gnum, waux, saux, pg = pl.pallas_call(
        _tc_d1,
        grid=(N // NB,),
        in_specs=[
            pl.BlockSpec((1, 1, NB), lambda i: (i, 0, 0)),
            pl.BlockSpec((1, 1, NB), lambda i: (i, 0, 0)),
            pl.BlockSpec((NB, NI), lambda i: (i, 0)),
            pl.BlockSpec((NB, KP), lambda i: (i, 0)),
        ],
        out_specs=[
            pl.BlockSpec((G, NI), lambda i: (0, 0)),
            pl.BlockSpec((G, NI), lambda i: (0, 0)),
            pl.BlockSpec((G, NI), lambda i: (0, 0)),
            pl.BlockSpec((G, KP), lambda i: (0, 0)),
        ],
        out_shape=[
            jax.ShapeDtypeStruct((G, NI), jnp.float32),
            jax.ShapeDtypeStruct((G, NI), jnp.float32),
            jax.ShapeDtypeStruct((G, NI), jnp.float32),
            jax.ShapeDtypeStruct((G, KP), jnp.float32),
        ],
    )(xb_3d, iw_3d, h, phi)

    theta, kl1b = pl.pallas_call(
        _tc_d2,
        out_shape=[
            jax.ShapeDtypeStruct((G, KP), jnp.float32),
            jax.ShapeDtypeStruct((G, 8), jnp.float32),
        ],
    )(gnum, waux, wmu_p, bmu_p, wlv_p, blv_p, eps_p)

    VB = 512
    at, bt, mzb = pl.pallas_call(
        _tc_f,
        grid=(VP // VB,),
        in_specs=[
            pl.BlockSpec((VB, NI), lambda i: (i, 0)),
            pl.BlockSpec((KP, NI), lambda i: (0, 0)),
            pl.BlockSpec((KP, NI), lambda i: (0, 0)),
            pl.BlockSpec((KP, NI), lambda i: (0, 0)),
        ],
        out_specs=[
            pl.BlockSpec((VB, KP), lambda i: (i, 0)),
            pl.BlockSpec((VB, KP), lambda i: (i, 0)),
            pl.BlockSpec((8, KP), lambda i: (0, 0)),
        ],
        out_shape=[
            jax.ShapeDtypeStruct((VP, KP), jnp.float32),
            jax.ShapeDtypeStruct((VP, KP), jnp.float32),
            jax.ShapeDtypeStruct((8, KP), jnp.float32),
        ],
    )(wvb_p, tvp, t1p, t2p)

    sega, segb = pl.pallas_call(
        _tc_e,
        grid=(N // NB,),
        in_specs=[
            pl.BlockSpec((1, 1, NB), lambda i: (i, 0, 0)),
            pl.BlockSpec((1, 1, NB), lambda i: (i, 0, 0)),
            pl.BlockSpec((NB, KP), lambda i: (i, 0)),
            pl.BlockSpec((NB, KP), lambda i: (i, 0)),
            pl.BlockSpec((NB, NI), lambda i: (i, 0)),
            pl.BlockSpec((KP, NI), lambda i: (0, 0)),
            pl.BlockSpec((G, KP), lambda i: (0, 0)),
            pl.BlockSpec((G, KP), lambda i: (0, 0)),
            pl.BlockSpec((8, KP), lambda i: (0, 0)),
        ],
        out_specs=[
            pl.BlockSpec((G, 8), lambda i: (0, 0)),
            pl.BlockSpec((G, 8), lambda i: (0, 0)),
        ],
        out_shape=[
            jax.ShapeDtypeStruct((G, 8), jnp.float32),
            jax.ShapeDtypeStruct((G, 8), jnp.float32),
        ],
    )(xb_3d, iw_3d, phi, qn, h0b, tvp, theta, pg, mzb)

    phi1, qpe0, qnui, phiuj, a100, b100 = sc2(
        phi, qp, qn, at, bt, e0, e1, ui, uj, we0p, we1p)

    WB = 512
    mze = pl.pallas_call(
        _tc_g,
        grid=(EWP // WB,),
        in_specs=[
            pl.BlockSpec((WB, KP), lambda i: (i, 0)),
            pl.BlockSpec((WB, KP), lambda i: (i, 0)),
        ],
        out_specs=pl.BlockSpec((8, KP), lambda i: (0, 0)),
        out_shape=jax.ShapeDtypeStruct((8, KP), jnp.float32),
    )(a100, b100)

    sval, kz0, same = sc3(a100, b100, edge_id, kz, e0, e1)

    scal = jnp.stack(
        [valid, gi.astype(jnp.float32), edge_id_batch.astype(jnp.float32),
         kz0.astype(jnp.float32), same, sval, _f32(edge_w),
         jnp.zeros((E,), jnp.float32)], axis=1)

    EB2 = 512
    out1, zew, bes = pl.pallas_call(
        _tc_h,
        grid=(E // EB2,),
        in_specs=[
            pl.BlockSpec((EB2, KP), lambda i: (i, 0)),
            pl.BlockSpec((EB2, KP), lambda i: (i, 0)),
            pl.BlockSpec((EB2, KP), lambda i: (i, 0)),
            pl.BlockSpec((EB2, KP), lambda i: (i, 0)),
            pl.BlockSpec((EB2, 8), lambda i: (i, 0)),
        ],
        out_specs=[
            pl.BlockSpec((G, 8), lambda i: (0, 0)),
            pl.BlockSpec((G, KP), lambda i: (0, 0)),
            pl.BlockSpec((G, KP), lambda i: (0, 0)),
        ],
        out_shape=[
            jax.ShapeDtypeStruct((G, 8), jnp.float32),
            jax.ShapeDtypeStruct((G, KP), jnp.float32),
            jax.ShapeDtypeStruct((G, KP), jnp.float32),
        ],
    )(qpe0, phi1, qnui, phiuj, scal)

    out = pl.pallas_call(
        _tc_i,
        out_shape=jax.ShapeDtypeStruct((8, NI), jnp.float32),
    )(out1, sega, segb, kl1b, saux, zew, bes, mze)

    return (out[0, 0], out[1, 0], out[2, 0], out[3, 0], out[4, 0],
            out[5, 0], out[6, 0])


# speedup vs baseline: 3.7863x; 1.0406x over previous
import functools
import jax
import jax.numpy as jnp
from jax import lax
from jax.experimental import pallas as pl
from jax.experimental.pallas import tpu as pltpu
from jax.experimental.pallas import tpu_sc as plsc

N = 4096
E = 65536
V = 50000
NI = 128
K = 50
KP = 128
G = 64
EW = 100000
VP = 50176
EWP = 100352
TEMP = 0.5

NC = 2
NS = 16
NW = NC * NS
EPW = E // NW
NPW = N // NW
WPW = EWP // NW

def _wid():
    return lax.axis_index("s") * NC + lax.axis_index("c")


_SC1_TYPES = dict(
    out_type=[
        jax.ShapeDtypeStruct((N, NI), jnp.float32),
        jax.ShapeDtypeStruct((N, NI), jnp.float32),
        jax.ShapeDtypeStruct((NC, N, NI), jnp.float32),
        jax.ShapeDtypeStruct((E,), jnp.int32),
        jax.ShapeDtypeStruct((E,), jnp.int32),
        jax.ShapeDtypeStruct((E,), jnp.int32),
        jax.ShapeDtypeStruct((E,), jnp.float32),
    ],
    scratch_types=[
        pltpu.VMEM((N,), jnp.int32),
        pltpu.VMEM((N,), jnp.int32),
        pltpu.VMEM((NPW,), jnp.int32),
        pltpu.VMEM((512, NI), jnp.float32),
        pltpu.VMEM((EPW,), jnp.int32),
        pltpu.VMEM((512,), jnp.int32),
        pltpu.VMEM((512,), jnp.int32),
        pltpu.VMEM((512,), jnp.int32),
        pltpu.VMEM((512,), jnp.int32),
        pltpu.VMEM((EPW,), jnp.float32),
        pltpu.VMEM((EPW,), jnp.int32),
        pltpu.VMEM((EPW,), jnp.int32),
        pltpu.VMEM((EPW,), jnp.int32),
        pltpu.VMEM((EPW,), jnp.int32),
        pltpu.VMEM((EPW,), jnp.int32),
        pltpu.VMEM((EPW,), jnp.float32),
        pltpu.VMEM_SHARED((N, NI), jnp.float32),
        pltpu.SemaphoreType.DMA,
    ],
)


def _sc1(wv_h, wvb_h, idxx_h, e0_h, e1_h, ew_h, zinit_h, skey_h, xb_h,
         h0_o, h0b_o, msg_o, ui_o, uj_o, gi_o, val_o,
         idxx_v, xb_v, idxn_v, rowbuf, e0_v, e1c0, e1c1, e1c2, e1c3,
         ew_v, idx2_v, sk_v,
         ui_v, uj_v, gi_v, val_v, acc_sh, sem):
    cid = lax.axis_index("c")
    sid = lax.axis_index("s")
    wid = _wid()
    base_n = wid * NPW
    base_e = wid * EPW
    @pl.when(sid == 0)
    def _():
        pltpu.sync_copy(zinit_h, acc_sh)
    plsc.subcore_barrier()
    pltpu.sync_copy(idxx_h.at[pl.ds(base_n, NPW)], idxn_v)
    pltpu.async_copy(wv_h.at[idxn_v], rowbuf.at[pl.ds(0, NPW)], sem).wait()
    pltpu.sync_copy(rowbuf.at[pl.ds(0, NPW)], h0_o.at[pl.ds(base_n, NPW)])
    pltpu.async_copy(wvb_h.at[idxn_v], rowbuf.at[pl.ds(0, NPW)], sem).wait()
    pltpu.sync_copy(rowbuf.at[pl.ds(0, NPW)], h0b_o.at[pl.ds(base_n, NPW)])
    pltpu.sync_copy(idxx_h, idxx_v)
    pltpu.sync_copy(xb_h, xb_v)
    pltpu.sync_copy(e0_h.at[pl.ds(base_e, EPW)], e0_v)

    def body_i2(i, _):
        ev = e0_v[pl.ds(i * 16, 16)]
        idx2_v[pl.ds(i * 16, 16)] = plsc.load_gather(idxx_v, [ev])
        return 0
    lax.fori_loop(0, EPW // 16, body_i2, 0)
    pltpu.sync_copy(ew_h.at[pl.ds(base_e, EPW)], ew_v)
    e1bufs = [e1c0, e1c1, e1c2, e1c3]
    for c in range(EPW // 512):
        pltpu.sync_copy(e1_h.at[pl.ds(base_e + c * 512, 512)], e1bufs[c])
        pltpu.async_copy(wv_h.at[idx2_v.at[pl.ds(c * 512, 512)]], rowbuf,
                         sem).wait()

        def body_w(j, _):
            wv16 = plsc.load_gather(ew_v, [jnp.full((16,), c * 512, jnp.int32)
                                           + j])
            for g in range(NI // 16):
                sl = pl.ds(g * 16, 16)
                rowbuf[j, sl] = rowbuf[j, sl] * wv16
            return 0
        lax.fori_loop(0, 512, body_w, 0)
        pltpu.sync_copy(rowbuf, acc_sh.at[e1bufs[c]], add=True)
    plsc.subcore_barrier()
    pltpu.sync_copy(acc_sh.at[pl.ds(sid * (N // NS), N // NS)],
                    msg_o.at[cid, pl.ds(sid * (N // NS), N // NS)])
    pltpu.sync_copy(skey_h.at[pl.ds(base_e, EPW)], sk_v)

    def body_uk(i, _):
        sl = pl.ds(i * 16, 16)
        kv = sk_v[sl]
        uiv = lax.shift_right_logical(kv, 12)
        ujv = lax.bitwise_and(kv, 4095)
        giv = plsc.load_gather(xb_v, [uiv])
        gjv = plsc.load_gather(xb_v, [ujv])
        ui_v[sl] = uiv
        uj_v[sl] = ujv
        gi_v[sl] = giv
        ok = jnp.logical_and(giv == gjv, uiv != ujv)
        val_v[sl] = jnp.where(ok, 1.0, 0.0).astype(jnp.float32)
        return 0
    lax.fori_loop(0, EPW // 16, body_uk, 0)
    pltpu.sync_copy(ui_v, ui_o.at[pl.ds(base_e, EPW)])
    pltpu.sync_copy(uj_v, uj_o.at[pl.ds(base_e, EPW)])
    pltpu.sync_copy(gi_v, gi_o.at[pl.ds(base_e, EPW)])
    pltpu.sync_copy(val_v, val_o.at[pl.ds(base_e, EPW)])


_SC2_TYPES = dict(
    out_type=[
        jax.ShapeDtypeStruct((E, KP), jnp.float32),
        jax.ShapeDtypeStruct((E, KP), jnp.float32),
        jax.ShapeDtypeStruct((E, KP), jnp.float32),
        jax.ShapeDtypeStruct((E, KP), jnp.float32),
        jax.ShapeDtypeStruct((EWP, KP), jnp.float32),
    ],
    scratch_types=[
        pltpu.VMEM((EPW,), jnp.int32),
        pltpu.VMEM((WPW,), jnp.int32),
        pltpu.VMEM((WPW,), jnp.int32),
        pltpu.VMEM((512, KP), jnp.float32),
        pltpu.VMEM((224, KP), jnp.float32),
        pltpu.SemaphoreType.DMA,
    ],
)


def _sc2(phi_h, qp_h, qn_h, at_h, bt_h, e0_h, e1_h, ui_h, uj_h,
         we0_h, we1_h,
         phi1_o, qpe0_o, qnui_o, phiuj_o, sab_o,
         idx_v, bigidx_v, bigidx2_v, rowbuf, bbuf, sem):
    wid = _wid()
    base_e = wid * EPW
    base_w = wid * WPW
    for idx_h, tab_h, out_o in ((e1_h, phi_h, phi1_o), (e0_h, qp_h, qpe0_o),
                                (ui_h, qn_h, qnui_o), (uj_h, phi_h, phiuj_o)):
        pltpu.sync_copy(idx_h.at[pl.ds(base_e, EPW)], idx_v)
        for c in range(EPW // 512):
            pltpu.async_copy(tab_h.at[idx_v.at[pl.ds(c * 512, 512)]], rowbuf,
                             sem).wait()
            pltpu.sync_copy(rowbuf, out_o.at[pl.ds(base_e + c * 512, 512)])
    pltpu.sync_copy(we0_h.at[pl.ds(base_w, WPW)], bigidx_v)
    pltpu.sync_copy(we1_h.at[pl.ds(base_w, WPW)], bigidx2_v)
    for c in range(WPW // 224):
        pltpu.async_copy(at_h.at[bigidx_v.at[pl.ds(c * 224, 224)]],
                         rowbuf.at[pl.ds(0, 224)], sem).wait()
        pltpu.async_copy(bt_h.at[bigidx2_v.at[pl.ds(c * 224, 224)]],
                         bbuf, sem).wait()

        def body_add(j, _):
            for g in range(KP // 16):
                sl = pl.ds(g * 16, 16)
                rowbuf[j, sl] = rowbuf[j, sl] + bbuf[j, sl]
            return 0
        lax.fori_loop(0, 224, body_add, 0)
        pltpu.sync_copy(rowbuf.at[pl.ds(0, 224)],
                        sab_o.at[pl.ds(base_w + c * 224, 224)])


_SC3_TYPES = dict(
    out_type=[
        jax.ShapeDtypeStruct((E,), jnp.float32),
        jax.ShapeDtypeStruct((E,), jnp.int32),
        jax.ShapeDtypeStruct((E,), jnp.float32),
    ],
    scratch_types=[
        pltpu.VMEM((N,), jnp.int32),
        pltpu.VMEM((EPW,), jnp.int32),
        pltpu.VMEM((EPW,), jnp.int32),
        pltpu.VMEM((EPW,), jnp.int32),
        pltpu.VMEM((EPW,), jnp.int32),
        pltpu.VMEM((EPW,), jnp.float32),
        pltpu.VMEM((EPW,), jnp.float32),
        pltpu.VMEM((512, KP), jnp.float32),
        pltpu.SemaphoreType.DMA,
    ],
)


def _sc3(sab_h, eid_h, kz_h, e0_h, e1_h,
         sval_o, kz0_o, same_o,
         kz_v, eid_v, e0_v, e1_v, kz0_v, sv_v, same_v, rowbuf, sem):
    wid = _wid()
    base_e = wid * EPW
    pltpu.sync_copy(kz_h, kz_v)
    pltpu.sync_copy(eid_h.at[pl.ds(base_e, EPW)], eid_v)
    pltpu.sync_copy(e0_h.at[pl.ds(base_e, EPW)], e0_v)
    pltpu.sync_copy(e1_h.at[pl.ds(base_e, EPW)], e1_v)

    def body_kz(i, _):
        sl = pl.ds(i * 16, 16)
        k0 = plsc.load_gather(kz_v, [e0_v[sl]])
        k1 = plsc.load_gather(kz_v, [e1_v[sl]])
        kz0_v[sl] = k0
        same_v[sl] = jnp.where(k0 == k1, 1.0, 0.0).astype(jnp.float32)
        return 0
    lax.fori_loop(0, EPW // 16, body_kz, 0)

    rows16 = lax.iota(jnp.int32, 16)
    for c in range(EPW // 512):
        pltpu.async_copy(sab_h.at[eid_v.at[pl.ds(c * 512, 512)]], rowbuf,
                         sem).wait()

        def body_pa(j, _):
            sl = pl.ds(c * 512 + j * 16, 16)
            va = plsc.load_gather(rowbuf, [rows16 + j * 16, kz0_v[sl]])
            sv_v[sl] = va
            return 0
        lax.fori_loop(0, 512 // 16, body_pa, 0)
    pltpu.sync_copy(sv_v, sval_o.at[pl.ds(base_e, EPW)])
    pltpu.sync_copy(kz0_v, kz0_o.at[pl.ds(base_e, EPW)])
    pltpu.sync_copy(same_v, same_o.at[pl.ds(base_e, EPW)])


@functools.lru_cache(maxsize=1)
def _sc_kernels():
    mesh = plsc.VectorSubcoreMesh(core_axis_name="c", subcore_axis_name="s")
    cp = pltpu.CompilerParams(needs_layout_passes=False)
    sc1 = pl.kernel(_sc1, mesh=mesh, compiler_params=cp, **_SC1_TYPES)
    sc2 = pl.kernel(_sc2, mesh=mesh, compiler_params=cp, **_SC2_TYPES)
    sc3 = pl.kernel(_sc3, mesh=mesh, compiler_params=cp, **_SC3_TYPES)
    return sc1, sc2, sc3


def _tc_a(tvcat_ref, wt_ref, lpw_ref, lnw_ref):
    tv = jnp.dot(tvcat_ref[...], wt_ref[...].T,
                 preferred_element_type=jnp.float32)
    s = jnp.dot(tv, tv.T, preferred_element_type=jnp.float32)
    wm = jnp.clip(jax.nn.sigmoid(s), 1e-6, 1.0 - 1e-6)
    r = lax.broadcasted_iota(jnp.int32, (KP, KP), 0)
    c = lax.broadcasted_iota(jnp.int32, (KP, KP), 1)
    mask = jnp.where(jnp.logical_and(r < K, c < K), 1.0, 0.0)
    lpw_ref[...] = jnp.log(wm) * mask
    lnw_ref[...] = jnp.log(1.0 - wm) * mask


def _tc_c(h0_ref, msga_ref, msgb_ref, wenc_ref, benc_ref, wphi_ref, bphi_ref,
          lpw_ref, lnw_ref, gumb_ref,
          h_ref, phi_ref, qp_ref, qn_ref, kz_ref):
    x = h0_ref[...] + msga_ref[...] + msgb_ref[...]
    h = jax.nn.relu(jnp.dot(x, wenc_ref[...],
                            preferred_element_type=jnp.float32) + benc_ref[...])
    h_ref[...] = h
    logits = jnp.dot(h, wphi_ref[...],
                     preferred_element_type=jnp.float32) + bphi_ref[...]
    m = jnp.max(logits, axis=1, keepdims=True)
    ex = jnp.exp(logits - m)
    phi = ex / jnp.sum(ex, axis=1, keepdims=True)
    phi_ref[...] = phi
    qp_ref[...] = jnp.dot(phi, lpw_ref[...], preferred_element_type=jnp.float32)
    qn_ref[...] = jnp.dot(phi, lnw_ref[...], preferred_element_type=jnp.float32)
    gl = jnp.log(phi + 1e-20) + gumb_ref[...]
    gm = jnp.max(gl, axis=1, keepdims=True)
    iota_k = lax.broadcasted_iota(jnp.int32, gl.shape, 1)
    cand = jnp.where(gl >= gm, iota_k, jnp.int32(10**9))
    kz = jnp.min(cand, axis=1)
    kz_ref[...] = jnp.reshape(kz, (1, 1, kz.shape[0]))


def _tc_d1(xb_ref, iw_ref, h_ref, phi_ref,
           gnum_ref, waux_ref, saux_ref, pg_ref):
    pid = pl.program_id(0)

    @pl.when(pid == 0)
    def _():
        gnum_ref[...] = jnp.zeros_like(gnum_ref)
        waux_ref[...] = jnp.zeros_like(waux_ref)
        saux_ref[...] = jnp.zeros_like(saux_ref)
        pg_ref[...] = jnp.zeros_like(pg_ref)
    xb = xb_ref[0]
    iota_g = lax.broadcasted_iota(jnp.int32, (G, xb.shape[1]), 0)
    geb = jnp.where(xb == iota_g, 1.0, 0.0)
    gw = geb * iw_ref[0]
    ones = jnp.ones((xb.shape[1], NI), jnp.float32)
    gnum_ref[...] += jnp.dot(gw, h_ref[...], preferred_element_type=jnp.float32)
    waux_ref[...] += jnp.dot(gw, ones, preferred_element_type=jnp.float32)
    saux_ref[...] += jnp.dot(geb, ones, preferred_element_type=jnp.float32)
    pg_ref[...] += jnp.dot(geb, phi_ref[...],
                           preferred_element_type=jnp.float32)


def _tc_d2(gnum_ref, waux_ref, wmu_ref, bmu_ref, wlv_ref, blv_ref, eps_ref,
           theta_ref, kl1_ref):
    g = gnum_ref[...] / (waux_ref[:, 0:1] + 1e-10)
    mu = jnp.dot(g, wmu_ref[...], preferred_element_type=jnp.float32) + bmu_ref[...]
    lv = jnp.dot(g, wlv_ref[...], preferred_element_type=jnp.float32) + blv_ref[...]
    kl1 = 0.5 * jnp.sum(mu * mu + jnp.exp(lv) - lv - 1.0, axis=1, keepdims=True)
    kl1_ref[...] = jnp.concatenate([kl1, jnp.zeros((G, 7), jnp.float32)], axis=1)
    t = mu + eps_ref[...] * jnp.exp(0.5 * lv)
    iota_k = lax.broadcasted_iota(jnp.int32, t.shape, 1)
    t = jnp.where(iota_k < K, t, -1e30)
    tm = jnp.max(t, axis=1, keepdims=True)
    te = jnp.exp(t - tm)
    theta_ref[...] = te / jnp.sum(te, axis=1, keepdims=True)


def _tc_e(xb_ref, iw_ref, phi_ref, qn_ref, h0b_ref, tvp_ref, theta_ref,
          pg_ref, mzb_ref, sega_ref, segb_ref):
    pid = pl.program_id(0)

    @pl.when(pid == 0)
    def _():
        sega_ref[...] = jnp.zeros_like(sega_ref)
        segb_ref[...] = jnp.zeros_like(segb_ref)
    xb = xb_ref[0]
    iota_g = lax.broadcasted_iota(jnp.int32, (G, xb.shape[1]), 0)
    geb = jnp.where(xb == iota_g, 1.0, 0.0)
    phi = phi_ref[...]
    thx = lax.dot_general(geb, theta_ref[...], (((0,), (0,)), ((), ())),
                          preferred_element_type=jnp.float32)
    kl2n = jnp.sum(phi * jnp.log(phi / (thx + 1e-10) + 1e-10), axis=1)
    tlogit = lax.dot_general(h0b_ref[...], tvp_ref[...], (((1,), (1,)), ((), ())),
                             preferred_element_type=jnp.float32)
    logzb = mzb_ref[2:3, :]
    beta_s = jnp.exp(tlogit - logzb)
    rwn = jnp.sum(phi * jnp.log(beta_s + 1e-6), axis=1)
    pgath = lax.dot_general(geb, pg_ref[...], (((0,), (0,)), ((), ())),
                            preferred_element_type=jnp.float32)
    n1n = jnp.sum(qn_ref[...] * (pgath - phi), axis=1)
    nb = kl2n.shape[0]
    zeros6 = jnp.zeros((nb, 6), jnp.float32)
    s2 = jnp.concatenate([kl2n[:, None], rwn[:, None], zeros6], axis=1)
    s1 = jnp.concatenate([n1n[:, None], zeros6, jnp.zeros((nb, 1), jnp.float32)],
                         axis=1)
    gw = geb * iw_ref[0]
    sega_ref[...] += jnp.dot(gw, s2, preferred_element_type=jnp.float32)
    segb_ref[...] += jnp.dot(geb, s1, preferred_element_type=jnp.float32)


def _tc_f(wvb_ref, tvp_ref, t1_ref, t2_ref, at_ref, bt_ref, mzb_ref):
    pid = pl.program_id(0)

    @pl.when(pid == 0)
    def _():
        mzb_ref[...] = jnp.zeros_like(mzb_ref)
        mzb_ref[0:1, :] = jnp.full((1, KP), -1e30, jnp.float32)
    wvb = wvb_ref[...]
    at_ref[...] = lax.dot_general(wvb, t1_ref[...], (((1,), (1,)), ((), ())),
                                  preferred_element_type=jnp.float32)
    bt_ref[...] = lax.dot_general(wvb, t2_ref[...], (((1,), (1,)), ((), ())),
                                  preferred_element_type=jnp.float32)
    st = lax.dot_general(wvb, tvp_ref[...], (((1,), (1,)), ((), ())),
                         preferred_element_type=jnp.float32)
    r = lax.broadcasted_iota(jnp.int32, st.shape, 0) + pid * st.shape[0]
    st = jnp.where(r < V, st, -1e30)
    bm = jnp.max(st, axis=0, keepdims=True)
    m_old = mzb_ref[0:1, :]
    s_old = mzb_ref[1:2, :]
    m_new = jnp.maximum(m_old, bm)
    s_new = s_old * jnp.exp(m_old - m_new) + jnp.sum(jnp.exp(st - m_new),
                                                     axis=0, keepdims=True)
    mzb_ref[0:1, :] = m_new
    mzb_ref[1:2, :] = s_new

    @pl.when(pid == pl.num_programs(0) - 1)
    def _():
        mzb_ref[2:3, :] = m_new + jnp.log(s_new)


def _tc_g(a_ref, mze_ref):
    pid = pl.program_id(0)

    @pl.when(pid == 0)
    def _():
        mze_ref[...] = jnp.zeros_like(mze_ref)
        mze_ref[0:1, :] = jnp.full((1, KP), -1e30, jnp.float32)
    s = a_ref[...]
    r = lax.broadcasted_iota(jnp.int32, s.shape, 0) + pid * s.shape[0]
    s = jnp.where(r < EW, s, -1e30)
    bm = jnp.max(s, axis=0, keepdims=True)
    m_old = mze_ref[0:1, :]
    s_old = mze_ref[1:2, :]
    m_new = jnp.maximum(m_old, bm)
    s_new = s_old * jnp.exp(m_old - m_new) + jnp.sum(jnp.exp(s - m_new),
                                                     axis=0, keepdims=True)
    mze_ref[0:1, :] = m_new
    mze_ref[1:2, :] = s_new

    @pl.when(pid == pl.num_programs(0) - 1)
    def _():
        mze_ref[2:3, :] = m_new + jnp.log(s_new)


def _tc_h(qpe0_ref, phi1_ref, qnui_ref, phiuj_ref, scal_ref,
          out1_ref, zew_ref, bes_ref):
    pid = pl.program_id(0)

    @pl.when(pid == 0)
    def _():
        out1_ref[...] = jnp.zeros_like(out1_ref)
        zew_ref[...] = jnp.zeros_like(zew_ref)
        bes_ref[...] = jnp.zeros_like(bes_ref)
    scal = scal_ref[...]
    nb = scal.shape[0]
    valid = scal[:, 0:1]
    pe = jnp.sum(qpe0_ref[...] * phi1_ref[...], axis=1, keepdims=True)
    nev = jnp.sum(qnui_ref[...] * phiuj_ref[...], axis=1, keepdims=True) * valid
    ones = jnp.ones((nb, 1), jnp.float32)
    zcol = jnp.zeros((nb, 1), jnp.float32)
    iota_g = lax.broadcasted_iota(jnp.int32, (nb, G), 1)
    iota_k = lax.broadcasted_iota(jnp.int32, (nb, KP), 1)
    gebt = jnp.where(scal[:, 2:3].astype(jnp.int32) == iota_g, 1.0, 0.0)
    ggit = jnp.where(scal[:, 1:2].astype(jnp.int32) == iota_g, 1.0, 0.0)
    k1h = jnp.where(scal[:, 3:4].astype(jnp.int32) == iota_k, 1.0, 0.0)
    zw = scal[:, 6:7] * scal[:, 4:5]
    bv = zw * jnp.exp(scal[:, 5:6])
    s6a = jnp.concatenate([pe, ones, zcol, zcol, zcol, zcol, zcol, zcol], axis=1)
    s6b = jnp.concatenate([zcol, zcol, nev, valid, zcol, zcol, zcol, zcol],
                          axis=1)
    out1_ref[...] += (
        lax.dot_general(gebt, s6a, (((0,), (0,)), ((), ())),
                        preferred_element_type=jnp.float32)
        + lax.dot_general(ggit, s6b, (((0,), (0,)), ((), ())),
                          preferred_element_type=jnp.float32))
    zew_ref[...] += lax.dot_general(gebt, k1h * zw, (((0,), (0,)), ((), ())),
                                    preferred_element_type=jnp.float32)
    bes_ref[...] += lax.dot_general(gebt, k1h * bv, (((0,), (0,)), ((), ())),
                                    preferred_element_type=jnp.float32)


def _tc_i(out1_ref, sega_ref, segb_ref, kl1_ref, saux_ref, zew_ref, bes_ref,
          mze_ref, out_ref):
    out1 = out1_ref[...]
    p_edge = out1[:, 0]
    np_ = out1[:, 1]
    necorr = out1[:, 2]
    cnt = out1[:, 3]
    kl2 = sega_ref[:, 0]
    recon_word = -sega_ref[:, 1]
    n1g = segb_ref[:, 0]
    kl1 = kl1_ref[:, 0]
    sizes = saux_ref[:, 0]
    n_edge1 = n1g - necorr
    nn = sizes * (sizes - 1.0) - cnt
    recon_structure = -(p_edge + n_edge1 / (nn + 1e-6) * np_)
    logze = mze_ref[2:3, :]
    bes = bes_ref[...] * jnp.exp(-logze)
    lr = (jnp.log(jnp.clip(bes, 1e-10, None))
          - jnp.log(jnp.clip(zew_ref[...], 1e-10, None)))
    recon_edge = -jnp.sum(lr, axis=1)
    loss = recon_edge + recon_word + kl1 + kl2 + recon_structure
    vals = [loss, recon_word, recon_edge, recon_structure, p_edge, kl1, kl2]
    r = lax.broadcasted_iota(jnp.int32, (8, NI), 0)
    c = lax.broadcasted_iota(jnp.int32, (8, NI), 1)
    acc = jnp.zeros((8, NI), jnp.float32)
    for i, v in enumerate(vals):
        mv = jnp.sum(v) / G
        acc = acc + jnp.where(jnp.logical_and(r == i, c == 0), mv, 0.0)
    out_ref[...] = acc


def _f32(x):
    return jnp.asarray(x, jnp.float32)


def kernel(idx_x, x_batch, idx_w, edge_w, edge_id, edge_id_batch, edge_index,
           whole_edge, word_vec, word_vec_beta, topic_vec, topic_edge_vec,
           W_topic, W_enc, b_enc, W_phi, b_phi, W_mu, b_mu, W_lv, b_lv):
    key = jax.random.key(42)
    idx_x = idx_x.astype(jnp.int32)
    x_batch = x_batch.astype(jnp.int32)
    edge_id = edge_id.astype(jnp.int32)
    edge_id_batch = edge_id_batch.astype(jnp.int32)
    e0 = edge_index[0].astype(jnp.int32)
    e1 = edge_index[1].astype(jnp.int32)
    we0 = whole_edge[0].astype(jnp.int32)
    we1 = whole_edge[1].astype(jnp.int32)
    word_vec = _f32(word_vec)
    wvb = _f32(word_vec_beta)

    skey = jnp.sort(e0 * N + e1)
    uniq = jnp.concatenate(
        [jnp.ones((1,), jnp.float32),
         (skey[1:] != skey[:-1]).astype(jnp.float32)])

    tvcat = jnp.zeros((KP, 3 * NI), jnp.float32).at[:K].set(
        jnp.concatenate([_f32(topic_vec), _f32(topic_edge_vec)], axis=-1))
    tvp = jnp.zeros((KP, NI), jnp.float32).at[:K].set(_f32(topic_vec))
    t1p = jnp.zeros((KP, NI), jnp.float32).at[:K].set(_f32(topic_edge_vec[:, :NI]))
    t2p = jnp.zeros((KP, NI), jnp.float32).at[:K].set(_f32(topic_edge_vec[:, NI:]))
    wphi_p = jnp.zeros((NI, KP), jnp.float32).at[:, :K].set(_f32(W_phi))
    bphi_p = jnp.full((1, KP), -1e30, jnp.float32).at[0, :K].set(_f32(b_phi))
    wmu_p = jnp.zeros((NI, KP), jnp.float32).at[:, :K].set(_f32(W_mu))
    bmu_p = jnp.zeros((1, KP), jnp.float32).at[0, :K].set(_f32(b_mu))
    wlv_p = jnp.zeros((NI, KP), jnp.float32).at[:, :K].set(_f32(W_lv))
    blv_p = jnp.zeros((1, KP), jnp.float32).at[0, :K].set(_f32(b_lv))
    benc = _f32(b_enc).reshape(1, NI)
    wvb_p = jnp.zeros((VP, NI), jnp.float32).at[:V].set(wvb)
    we0p = jnp.zeros((EWP,), jnp.int32).at[:EW].set(we0)
    we1p = jnp.zeros((EWP,), jnp.int32).at[:EW].set(we1)

    eps = jax.random.normal(jax.random.fold_in(key, 1), (G, K))
    eps_p = jnp.zeros((G, KP), jnp.float32).at[:, :K].set(eps)
    u = jax.random.uniform(jax.random.fold_in(key, 2), (1, N, K),
                           minval=1e-10, maxval=1.0)
    gumb = -jnp.log(-jnp.log(u))[0]
    gumb_p = jnp.full((N, KP), -1e30, jnp.float32).at[:, :K].set(gumb)

    sc1, sc2, sc3 = _sc_kernels()
    zinit = jnp.zeros((N, NI), jnp.float32)
    h0, h0b, msg2, ui, uj, gi, val0 = sc1(word_vec, wvb, idx_x, e0, e1,
                                          _f32(edge_w), zinit, skey, x_batch)
    valid = val0 * uniq

    lpw, lnw = pl.pallas_call(
        _tc_a,
        out_shape=[jax.ShapeDtypeStruct((KP, KP), jnp.float32)] * 2,
    )(tvcat, _f32(W_topic))

    NB = 512
    full64 = pl.BlockSpec((KP, KP), lambda i: (0, 0))
    h, phi, qp, qn, kz3 = pl.pallas_call(
        _tc_c,
        grid=(N // NB,),
        in_specs=[
            pl.BlockSpec((NB, NI), lambda i: (i, 0)),
            pl.BlockSpec((NB, NI), lambda i: (i, 0)),
            pl.BlockSpec((NB, NI), lambda i: (i, 0)),
            pl.BlockSpec((NI, NI), lambda i: (0, 0)),
            pl.BlockSpec((1, NI), lambda i: (0, 0)),
            pl.BlockSpec((NI, KP), lambda i: (0, 0)),
            pl.BlockSpec((1, KP), lambda i: (0, 0)),
            full64, full64,
            pl.BlockSpec((NB, KP), lambda i: (i, 0)),
        ],
        out_specs=[
            pl.BlockSpec((NB, NI), lambda i: (i, 0)),
            pl.BlockSpec((NB, KP), lambda i: (i, 0)),
            pl.BlockSpec((NB, KP), lambda i: (i, 0)),
            pl.BlockSpec((NB, KP), lambda i: (i, 0)),
            pl.BlockSpec((1, 1, NB), lambda i: (i, 0, 0)),
        ],
        out_shape=[
            jax.ShapeDtypeStruct((N, NI), jnp.float32),
            jax.ShapeDtypeStruct((N, KP), jnp.float32),
            jax.ShapeDtypeStruct((N, KP), jnp.float32),
            jax.ShapeDtypeStruct((N, KP), jnp.float32),
            jax.ShapeDtypeStruct((N // NB, 1, NB), jnp.int32),
        ],
    )(h0, msg2[0], msg2[1], _f32(W_enc), benc, wphi_p, bphi_p, lpw, lnw,
      gumb_p)
    kz = kz3.reshape(N)

    xb_3d = x_batch.reshape(N // NB, 1, NB)
    iw_3d = _f32(idx_w).reshape(N // NB, 1, NB)
    gnum, waux, saux, pg = pl.pallas_call(
        _tc_d1,
        grid=(N // NB,),
        in_specs=[
            pl.BlockSpec((1, 1, NB), lambda i: (i, 0, 0)),
            pl.BlockSpec((1, 1, NB), lambda i: (i, 0, 0)),
            pl.BlockSpec((NB, NI), lambda i: (i, 0)),
            pl.BlockSpec((NB, KP), lambda i: (i, 0)),
        ],
        out_specs=[
            pl.BlockSpec((G, NI), lambda i: (0, 0)),
            pl.BlockSpec((G, NI), lambda i: (0, 0)),
            pl.BlockSpec((G, NI), lambda i: (0, 0)),
            pl.BlockSpec((G, KP), lambda i: (0, 0)),
        ],
        out_shape=[
            jax.ShapeDtypeStruct((G, NI), jnp.float32),
            jax.ShapeDtypeStruct((G, NI), jnp.float32),
            jax.ShapeDtypeStruct((G, NI), jnp.float32),
            jax.ShapeDtypeStruct((G, KP), jnp.float32),
        ],
    )(xb_3d, iw_3d, h, phi)

    theta, kl1b = pl.pallas_call(
        _tc_d2,
        out_shape=[
            jax.ShapeDtypeStruct((G, KP), jnp.float32),
            jax.ShapeDtypeStruct((G, 8), jnp.float32),
        ],
    )(gnum, waux, wmu_p, bmu_p, wlv_p, blv_p, eps_p)

    VB = 512
    at, bt, mzb = pl.pallas_call(
        _tc_f,
        grid=(VP // VB,),
        in_specs=[
            pl.BlockSpec((VB, NI), lambda i: (i, 0)),
            pl.BlockSpec((KP, NI), lambda i: (0, 0)),
            pl.BlockSpec((KP, NI), lambda i: (0, 0)),
            pl.BlockSpec((KP, NI), lambda i: (0, 0)),
        ],
        out_specs=[
            pl.BlockSpec((VB, KP), lambda i: (i, 0)),
            pl.BlockSpec((VB, KP), lambda i: (i, 0)),
            pl.BlockSpec((8, KP), lambda i: (0, 0)),
        ],
        out_shape=[
            jax.ShapeDtypeStruct((VP, KP), jnp.float32),
            jax.ShapeDtypeStruct((VP, KP), jnp.float32),
            jax.ShapeDtypeStruct((8, KP), jnp.float32),
        ],
    )(wvb_p, tvp, t1p, t2p)

    sega, segb = pl.pallas_call(
        _tc_e,
        grid=(N // NB,),
        in_specs=[
            pl.BlockSpec((1, 1, NB), lambda i: (i, 0, 0)),
            pl.BlockSpec((1, 1, NB), lambda i: (i, 0, 0)),
            pl.BlockSpec((NB, KP), lambda i: (i, 0)),
            pl.BlockSpec((NB, KP), lambda i: (i, 0)),
            pl.BlockSpec((NB, NI), lambda i: (i, 0)),
            pl.BlockSpec((KP, NI), lambda i: (0, 0)),
            pl.BlockSpec((G, KP), lambda i: (0, 0)),
            pl.BlockSpec((G, KP), lambda i: (0, 0)),
            pl.BlockSpec((8, KP), lambda i: (0, 0)),
        ],
        out_specs=[
            pl.BlockSpec((G, 8), lambda i: (0, 0)),
            pl.BlockSpec((G, 8), lambda i: (0, 0)),
        ],
        out_shape=[
            jax.ShapeDtypeStruct((G, 8), jnp.float32),
            jax.ShapeDtypeStruct((G, 8), jnp.float32),
        ],
    )(xb_3d, iw_3d, phi, qn, h0b, tvp, theta, pg, mzb)

    phi1, qpe0, qnui, phiuj, sab = sc2(
        phi, qp, qn, at, bt, e0, e1, ui, uj, we0p, we1p)

    WB = 512
    mze = pl.pallas_call(
        _tc_g,
        grid=(EWP // WB,),
        in_specs=[
            pl.BlockSpec((WB, KP), lambda i: (i, 0)),
        ],
        out_specs=pl.BlockSpec((8, KP), lambda i: (0, 0)),
        out_shape=jax.ShapeDtypeStruct((8, KP), jnp.float32),
    )(sab)

    sval, kz0, same = sc3(sab, edge_id, kz, e0, e1)

    scal = jnp.stack(
        [valid, gi.astype(jnp.float32), edge_id_batch.astype(jnp.float32),
         kz0.astype(jnp.float32), same, sval, _f32(edge_w),
         jnp.zeros((E,), jnp.float32)], axis=1)

    EB2 = 512
    out1, zew, bes = pl.pallas_call(
        _tc_h,
        grid=(E // EB2,),
        in_specs=[
            pl.BlockSpec((EB2, KP), lambda i: (i, 0)),
            pl.BlockSpec((EB2, KP), lambda i: (i, 0)),
            pl.BlockSpec((EB2, KP), lambda i: (i, 0)),
            pl.BlockSpec((EB2, KP), lambda i: (i, 0)),
            pl.BlockSpec((EB2, 8), lambda i: (i, 0)),
        ],
        out_specs=[
            pl.BlockSpec((G, 8), lambda i: (0, 0)),
            pl.BlockSpec((G, KP), lambda i: (0, 0)),
            pl.BlockSpec((G, KP), lambda i: (0, 0)),
        ],
        out_shape=[
            jax.ShapeDtypeStruct((G, 8), jnp.float32),
            jax.ShapeDtypeStruct((G, KP), jnp.float32),
            jax.ShapeDtypeStruct((G, KP), jnp.float32),
        ],
    )(qpe0, phi1, qnui, phiuj, scal)

    out = pl.pallas_call(
        _tc_i,
        out_shape=jax.ShapeDtypeStruct((8, NI), jnp.float32),
    )(out1, sega, segb, kl1b, saux, zew, bes, mze)

    return (out[0, 0], out[1, 0], out[2, 0], out[3, 0], out[4, 0],
            out[5, 0], out[6, 0])


# speedup vs baseline: 3.8010x; 1.0039x over previous
import functools
import jax
import jax.numpy as jnp
from jax import lax
from jax.experimental import pallas as pl
from jax.experimental.pallas import tpu as pltpu
from jax.experimental.pallas import tpu_sc as plsc

N = 4096
E = 65536
V = 50000
NI = 128
K = 50
KP = 128
G = 64
EW = 100000
VP = 50176
EWP = 100352
TEMP = 0.5

NC = 2
NS = 16
NW = NC * NS
EPW = E // NW
NPW = N // NW
WPW = EWP // NW

def _wid():
    return lax.axis_index("s") * NC + lax.axis_index("c")


_SC1_TYPES = dict(
    out_type=[
        jax.ShapeDtypeStruct((N, NI), jnp.float32),
        jax.ShapeDtypeStruct((N, NI), jnp.float32),
        jax.ShapeDtypeStruct((NC, N, NI), jnp.float32),
        jax.ShapeDtypeStruct((E,), jnp.int32),
        jax.ShapeDtypeStruct((E,), jnp.int32),
        jax.ShapeDtypeStruct((E,), jnp.int32),
        jax.ShapeDtypeStruct((E,), jnp.float32),
    ],
    scratch_types=[
        pltpu.VMEM((N,), jnp.int32),
        pltpu.VMEM((N,), jnp.int32),
        pltpu.VMEM((NPW,), jnp.int32),
        pltpu.VMEM((512, NI), jnp.float32),
        pltpu.VMEM((EPW,), jnp.int32),
        pltpu.VMEM((512,), jnp.int32),
        pltpu.VMEM((512,), jnp.int32),
        pltpu.VMEM((512,), jnp.int32),
        pltpu.VMEM((512,), jnp.int32),
        pltpu.VMEM((EPW,), jnp.float32),
        pltpu.VMEM((EPW,), jnp.int32),
        pltpu.VMEM((EPW,), jnp.int32),
        pltpu.VMEM((EPW,), jnp.int32),
        pltpu.VMEM((EPW,), jnp.int32),
        pltpu.VMEM((EPW,), jnp.int32),
        pltpu.VMEM((EPW,), jnp.float32),
        pltpu.VMEM_SHARED((N, NI), jnp.float32),
        pltpu.SemaphoreType.DMA,
    ],
)


def _sc1(wv_h, wvb_h, idxx_h, e0_h, e1_h, ew_h, zinit_h, skey_h, xb_h,
         h0_o, h0b_o, msg_o, ui_o, uj_o, gi_o, val_o,
         idxx_v, xb_v, idxn_v, rowbuf, e0_v, e1c0, e1c1, e1c2, e1c3,
         ew_v, idx2_v, sk_v,
         ui_v, uj_v, gi_v, val_v, acc_sh, sem):
    cid = lax.axis_index("c")
    sid = lax.axis_index("s")
    wid = _wid()
    base_n = wid * NPW
    base_e = wid * EPW
    @pl.when(sid == 0)
    def _():
        pltpu.sync_copy(zinit_h, acc_sh)
    plsc.subcore_barrier()
    pltpu.sync_copy(idxx_h.at[pl.ds(base_n, NPW)], idxn_v)
    pltpu.async_copy(wv_h.at[idxn_v], rowbuf.at[pl.ds(0, NPW)], sem).wait()
    pltpu.sync_copy(rowbuf.at[pl.ds(0, NPW)], h0_o.at[pl.ds(base_n, NPW)])
    pltpu.async_copy(wvb_h.at[idxn_v], rowbuf.at[pl.ds(0, NPW)], sem).wait()
    pltpu.sync_copy(rowbuf.at[pl.ds(0, NPW)], h0b_o.at[pl.ds(base_n, NPW)])
    pltpu.sync_copy(idxx_h, idxx_v)
    pltpu.sync_copy(xb_h, xb_v)
    pltpu.sync_copy(e0_h.at[pl.ds(base_e, EPW)], e0_v)

    def body_i2(i, _):
        ev = e0_v[pl.ds(i * 16, 16)]
        idx2_v[pl.ds(i * 16, 16)] = plsc.load_gather(idxx_v, [ev])
        return 0
    lax.fori_loop(0, EPW // 16, body_i2, 0)
    pltpu.sync_copy(ew_h.at[pl.ds(base_e, EPW)], ew_v)
    e1bufs = [e1c0, e1c1, e1c2, e1c3]
    for c in range(EPW // 512):
        pltpu.sync_copy(e1_h.at[pl.ds(base_e + c * 512, 512)], e1bufs[c])
        pltpu.async_copy(wv_h.at[idx2_v.at[pl.ds(c * 512, 512)]], rowbuf,
                         sem).wait()

        def body_w(j, _):
            wv16 = plsc.load_gather(ew_v, [jnp.full((16,), c * 512, jnp.int32)
                                           + j])
            for g in range(NI // 16):
                sl = pl.ds(g * 16, 16)
                rowbuf[j, sl] = rowbuf[j, sl] * wv16
            return 0
        lax.fori_loop(0, 512, body_w, 0)
        pltpu.sync_copy(rowbuf, acc_sh.at[e1bufs[c]], add=True)
    plsc.subcore_barrier()
    pltpu.sync_copy(acc_sh.at[pl.ds(sid * (N // NS), N // NS)],
                    msg_o.at[cid, pl.ds(sid * (N // NS), N // NS)])
    pltpu.sync_copy(skey_h.at[pl.ds(base_e, EPW)], sk_v)

    def body_uk(i, _):
        sl = pl.ds(i * 16, 16)
        kv = sk_v[sl]
        uiv = lax.shift_right_logical(kv, 12)
        ujv = lax.bitwise_and(kv, 4095)
        giv = plsc.load_gather(xb_v, [uiv])
        gjv = plsc.load_gather(xb_v, [ujv])
        ui_v[sl] = uiv
        uj_v[sl] = ujv
        gi_v[sl] = giv
        ok = jnp.logical_and(giv == gjv, uiv != ujv)
        val_v[sl] = jnp.where(ok, 1.0, 0.0).astype(jnp.float32)
        return 0
    lax.fori_loop(0, EPW // 16, body_uk, 0)
    pltpu.sync_copy(ui_v, ui_o.at[pl.ds(base_e, EPW)])
    pltpu.sync_copy(uj_v, uj_o.at[pl.ds(base_e, EPW)])
    pltpu.sync_copy(gi_v, gi_o.at[pl.ds(base_e, EPW)])
    pltpu.sync_copy(val_v, val_o.at[pl.ds(base_e, EPW)])


_SC2_TYPES = dict(
    out_type=[
        jax.ShapeDtypeStruct((E, KP), jnp.float32),
        jax.ShapeDtypeStruct((E, KP), jnp.float32),
        jax.ShapeDtypeStruct((E, KP), jnp.float32),
        jax.ShapeDtypeStruct((E, KP), jnp.float32),
        jax.ShapeDtypeStruct((EWP, KP), jnp.float32),
    ],
    scratch_types=[
        pltpu.VMEM((EPW,), jnp.int32),
        pltpu.VMEM((WPW,), jnp.int32),
        pltpu.VMEM((WPW,), jnp.int32),
        pltpu.VMEM((512, KP), jnp.float32),
        pltpu.VMEM((224, KP), jnp.float32),
        pltpu.SemaphoreType.DMA,
    ],
)


def _sc2(phi_h, qp_h, qn_h, at_h, bt_h, e0_h, e1_h, ui_h, uj_h,
         we0_h, we1_h,
         phi1_o, qpe0_o, qnui_o, phiuj_o, sab_o,
         idx_v, bigidx_v, bigidx2_v, rowbuf, bbuf, sem):
    wid = _wid()
    base_e = wid * EPW
    base_w = wid * WPW
    for idx_h, tab_h, out_o in ((e1_h, phi_h, phi1_o), (e0_h, qp_h, qpe0_o),
                                (ui_h, qn_h, qnui_o), (uj_h, phi_h, phiuj_o)):
        pltpu.sync_copy(idx_h.at[pl.ds(base_e, EPW)], idx_v)
        for c in range(EPW // 512):
            pltpu.async_copy(tab_h.at[idx_v.at[pl.ds(c * 512, 512)]], rowbuf,
                             sem).wait()
            pltpu.sync_copy(rowbuf, out_o.at[pl.ds(base_e + c * 512, 512)])
    pltpu.sync_copy(we0_h.at[pl.ds(base_w, WPW)], bigidx_v)
    pltpu.sync_copy(we1_h.at[pl.ds(base_w, WPW)], bigidx2_v)
    for c in range(WPW // 224):
        pltpu.async_copy(at_h.at[bigidx_v.at[pl.ds(c * 224, 224)]],
                         rowbuf.at[pl.ds(0, 224)], sem).wait()
        pltpu.async_copy(bt_h.at[bigidx2_v.at[pl.ds(c * 224, 224)]],
                         bbuf, sem).wait()

        def body_add(j, _):
            for g in range(KP // 16):
                sl = pl.ds(g * 16, 16)
                rowbuf[j, sl] = rowbuf[j, sl] + bbuf[j, sl]
            return 0
        lax.fori_loop(0, 224, body_add, 0)
        pltpu.sync_copy(rowbuf.at[pl.ds(0, 224)],
                        sab_o.at[pl.ds(base_w + c * 224, 224)])


_SC3_TYPES = dict(
    out_type=[
        jax.ShapeDtypeStruct((E,), jnp.float32),
        jax.ShapeDtypeStruct((E,), jnp.int32),
        jax.ShapeDtypeStruct((E,), jnp.float32),
    ],
    scratch_types=[
        pltpu.VMEM((N,), jnp.int32),
        pltpu.VMEM((EPW,), jnp.int32),
        pltpu.VMEM((EPW,), jnp.int32),
        pltpu.VMEM((EPW,), jnp.int32),
        pltpu.VMEM((EPW,), jnp.int32),
        pltpu.VMEM((EPW,), jnp.float32),
        pltpu.VMEM((EPW,), jnp.float32),
        pltpu.VMEM((512, KP), jnp.float32),
        pltpu.SemaphoreType.DMA,
    ],
)


def _sc3(sab_h, eid_h, kz_h, e0_h, e1_h,
         sval_o, kz0_o, same_o,
         kz_v, eid_v, e0_v, e1_v, kz0_v, sv_v, same_v, rowbuf, sem):
    wid = _wid()
    base_e = wid * EPW
    pltpu.sync_copy(kz_h, kz_v)
    pltpu.sync_copy(eid_h.at[pl.ds(base_e, EPW)], eid_v)
    pltpu.sync_copy(e0_h.at[pl.ds(base_e, EPW)], e0_v)
    pltpu.sync_copy(e1_h.at[pl.ds(base_e, EPW)], e1_v)

    def body_kz(i, _):
        sl = pl.ds(i * 16, 16)
        k0 = plsc.load_gather(kz_v, [e0_v[sl]])
        k1 = plsc.load_gather(kz_v, [e1_v[sl]])
        kz0_v[sl] = k0
        same_v[sl] = jnp.where(k0 == k1, 1.0, 0.0).astype(jnp.float32)
        return 0
    lax.fori_loop(0, EPW // 16, body_kz, 0)

    rows16 = lax.iota(jnp.int32, 16)
    for c in range(EPW // 512):
        pltpu.async_copy(sab_h.at[eid_v.at[pl.ds(c * 512, 512)]], rowbuf,
                         sem).wait()

        def body_pa(j, _):
            sl = pl.ds(c * 512 + j * 16, 16)
            va = plsc.load_gather(rowbuf, [rows16 + j * 16, kz0_v[sl]])
            sv_v[sl] = va
            return 0
        lax.fori_loop(0, 512 // 16, body_pa, 0)
    pltpu.sync_copy(sv_v, sval_o.at[pl.ds(base_e, EPW)])
    pltpu.sync_copy(kz0_v, kz0_o.at[pl.ds(base_e, EPW)])
    pltpu.sync_copy(same_v, same_o.at[pl.ds(base_e, EPW)])


@functools.lru_cache(maxsize=1)
def _sc_kernels():
    mesh = plsc.VectorSubcoreMesh(core_axis_name="c", subcore_axis_name="s")
    cp = pltpu.CompilerParams(needs_layout_passes=False)
    sc1 = pl.kernel(_sc1, mesh=mesh, compiler_params=cp, **_SC1_TYPES)
    sc2 = pl.kernel(_sc2, mesh=mesh, compiler_params=cp, **_SC2_TYPES)
    sc3 = pl.kernel(_sc3, mesh=mesh, compiler_params=cp, **_SC3_TYPES)
    return sc1, sc2, sc3


def _tc_c(h0_ref, msga_ref, msgb_ref, wenc_ref, benc_ref, wphi_ref, bphi_ref,
          tvcat_ref, wt_ref, gumb_ref,
          h_ref, phi_ref, qp_ref, qn_ref, kz_ref):
    tv = jnp.dot(tvcat_ref[...], wt_ref[...].T,
                 preferred_element_type=jnp.float32)
    s_tt = jnp.dot(tv, tv.T, preferred_element_type=jnp.float32)
    wm = jnp.clip(jax.nn.sigmoid(s_tt), 1e-6, 1.0 - 1e-6)
    rr = lax.broadcasted_iota(jnp.int32, (KP, KP), 0)
    cc = lax.broadcasted_iota(jnp.int32, (KP, KP), 1)
    mask_tt = jnp.where(jnp.logical_and(rr < K, cc < K), 1.0, 0.0)
    lpw = jnp.log(wm) * mask_tt
    lnw = jnp.log(1.0 - wm) * mask_tt
    x = h0_ref[...] + msga_ref[...] + msgb_ref[...]
    h = jax.nn.relu(jnp.dot(x, wenc_ref[...],
                            preferred_element_type=jnp.float32) + benc_ref[...])
    h_ref[...] = h
    logits = jnp.dot(h, wphi_ref[...],
                     preferred_element_type=jnp.float32) + bphi_ref[...]
    m = jnp.max(logits, axis=1, keepdims=True)
    ex = jnp.exp(logits - m)
    phi = ex / jnp.sum(ex, axis=1, keepdims=True)
    phi_ref[...] = phi
    qp_ref[...] = jnp.dot(phi, lpw, preferred_element_type=jnp.float32)
    qn_ref[...] = jnp.dot(phi, lnw, preferred_element_type=jnp.float32)
    gl = jnp.log(phi + 1e-20) + gumb_ref[...]
    gm = jnp.max(gl, axis=1, keepdims=True)
    iota_k = lax.broadcasted_iota(jnp.int32, gl.shape, 1)
    cand = jnp.where(gl >= gm, iota_k, jnp.int32(10**9))
    kz = jnp.min(cand, axis=1)
    kz_ref[...] = jnp.reshape(kz, (1, 1, kz.shape[0]))


def _tc_d1(xb_ref, iw_ref, h_ref, phi_ref,
           gnum_ref, waux_ref, saux_ref, pg_ref):
    pid = pl.program_id(0)

    @pl.when(pid == 0)
    def _():
        gnum_ref[...] = jnp.zeros_like(gnum_ref)
        waux_ref[...] = jnp.zeros_like(waux_ref)
        saux_ref[...] = jnp.zeros_like(saux_ref)
        pg_ref[...] = jnp.zeros_like(pg_ref)
    xb = xb_ref[0]
    iota_g = lax.broadcasted_iota(jnp.int32, (G, xb.shape[1]), 0)
    geb = jnp.where(xb == iota_g, 1.0, 0.0)
    gw = geb * iw_ref[0]
    ones = jnp.ones((xb.shape[1], NI), jnp.float32)
    gnum_ref[...] += jnp.dot(gw, h_ref[...], preferred_element_type=jnp.float32)
    waux_ref[...] += jnp.dot(gw, ones, preferred_element_type=jnp.float32)
    saux_ref[...] += jnp.dot(geb, ones, preferred_element_type=jnp.float32)
    pg_ref[...] += jnp.dot(geb, phi_ref[...],
                           preferred_element_type=jnp.float32)


def _tc_e(xb_ref, iw_ref, phi_ref, qn_ref, h0b_ref, tvp_ref,
          gnum_ref, waux_ref, wmu_ref, bmu_ref, wlv_ref, blv_ref, eps_ref,
          pg_ref, mzb_ref, sega_ref, segb_ref, kl1_ref):
    pid = pl.program_id(0)

    @pl.when(pid == 0)
    def _():
        sega_ref[...] = jnp.zeros_like(sega_ref)
        segb_ref[...] = jnp.zeros_like(segb_ref)
    g = gnum_ref[...] / (waux_ref[:, 0:1] + 1e-10)
    mu = jnp.dot(g, wmu_ref[...], preferred_element_type=jnp.float32) + bmu_ref[...]
    lv = jnp.dot(g, wlv_ref[...], preferred_element_type=jnp.float32) + blv_ref[...]
    kl1 = 0.5 * jnp.sum(mu * mu + jnp.exp(lv) - lv - 1.0, axis=1, keepdims=True)
    kl1_ref[...] = jnp.concatenate([kl1, jnp.zeros((G, 7), jnp.float32)], axis=1)
    t = mu + eps_ref[...] * jnp.exp(0.5 * lv)
    iota_kt = lax.broadcasted_iota(jnp.int32, t.shape, 1)
    t = jnp.where(iota_kt < K, t, -1e30)
    tm = jnp.max(t, axis=1, keepdims=True)
    te = jnp.exp(t - tm)
    theta = te / jnp.sum(te, axis=1, keepdims=True)
    xb = xb_ref[0]
    iota_g = lax.broadcasted_iota(jnp.int32, (G, xb.shape[1]), 0)
    geb = jnp.where(xb == iota_g, 1.0, 0.0)
    phi = phi_ref[...]
    thx = lax.dot_general(geb, theta, (((0,), (0,)), ((), ())),
                          preferred_element_type=jnp.float32)
    kl2n = jnp.sum(phi * jnp.log(phi / (thx + 1e-10) + 1e-10), axis=1)
    tlogit = lax.dot_general(h0b_ref[...], tvp_ref[...], (((1,), (1,)), ((), ())),
                             preferred_element_type=jnp.float32)
    logzb = mzb_ref[2:3, :]
    beta_s = jnp.exp(tlogit - logzb)
    rwn = jnp.sum(phi * jnp.log(beta_s + 1e-6), axis=1)
    pgath = lax.dot_general(geb, pg_ref[...], (((0,), (0,)), ((), ())),
                            preferred_element_type=jnp.float32)
    n1n = jnp.sum(qn_ref[...] * (pgath - phi), axis=1)
    nb = kl2n.shape[0]
    zeros6 = jnp.zeros((nb, 6), jnp.float32)
    s2 = jnp.concatenate([kl2n[:, None], rwn[:, None], zeros6], axis=1)
    s1 = jnp.concatenate([n1n[:, None], zeros6, jnp.zeros((nb, 1), jnp.float32)],
                         axis=1)
    gw = geb * iw_ref[0]
    sega_ref[...] += jnp.dot(gw, s2, preferred_element_type=jnp.float32)
    segb_ref[...] += jnp.dot(geb, s1, preferred_element_type=jnp.float32)


def _tc_f(wvb_ref, tvp_ref, t1_ref, t2_ref, at_ref, bt_ref, mzb_ref):
    pid = pl.program_id(0)

    @pl.when(pid == 0)
    def _():
        mzb_ref[...] = jnp.zeros_like(mzb_ref)
        mzb_ref[0:1, :] = jnp.full((1, KP), -1e30, jnp.float32)
    wvb = wvb_ref[...]
    at_ref[...] = lax.dot_general(wvb, t1_ref[...], (((1,), (1,)), ((), ())),
                                  preferred_element_type=jnp.float32)
    bt_ref[...] = lax.dot_general(wvb, t2_ref[...], (((1,), (1,)), ((), ())),
                                  preferred_element_type=jnp.float32)
    st = lax.dot_general(wvb, tvp_ref[...], (((1,), (1,)), ((), ())),
                         preferred_element_type=jnp.float32)
    r = lax.broadcasted_iota(jnp.int32, st.shape, 0) + pid * st.shape[0]
    st = jnp.where(r < V, st, -1e30)
    bm = jnp.max(st, axis=0, keepdims=True)
    m_old = mzb_ref[0:1, :]
    s_old = mzb_ref[1:2, :]
    m_new = jnp.maximum(m_old, bm)
    s_new = s_old * jnp.exp(m_old - m_new) + jnp.sum(jnp.exp(st - m_new),
                                                     axis=0, keepdims=True)
    mzb_ref[0:1, :] = m_new
    mzb_ref[1:2, :] = s_new

    @pl.when(pid == pl.num_programs(0) - 1)
    def _():
        mzb_ref[2:3, :] = m_new + jnp.log(s_new)


def _tc_g(a_ref, mze_ref):
    pid = pl.program_id(0)

    @pl.when(pid == 0)
    def _():
        mze_ref[...] = jnp.zeros_like(mze_ref)
        mze_ref[0:1, :] = jnp.full((1, KP), -1e30, jnp.float32)
    s = a_ref[...]
    r = lax.broadcasted_iota(jnp.int32, s.shape, 0) + pid * s.shape[0]
    s = jnp.where(r < EW, s, -1e30)
    bm = jnp.max(s, axis=0, keepdims=True)
    m_old = mze_ref[0:1, :]
    s_old = mze_ref[1:2, :]
    m_new = jnp.maximum(m_old, bm)
    s_new = s_old * jnp.exp(m_old - m_new) + jnp.sum(jnp.exp(s - m_new),
                                                     axis=0, keepdims=True)
    mze_ref[0:1, :] = m_new
    mze_ref[1:2, :] = s_new

    @pl.when(pid == pl.num_programs(0) - 1)
    def _():
        mze_ref[2:3, :] = m_new + jnp.log(s_new)


def _tc_h(qpe0_ref, phi1_ref, qnui_ref, phiuj_ref, scal_ref,
          out1_ref, zew_ref, bes_ref):
    pid = pl.program_id(0)

    @pl.when(pid == 0)
    def _():
        out1_ref[...] = jnp.zeros_like(out1_ref)
        zew_ref[...] = jnp.zeros_like(zew_ref)
        bes_ref[...] = jnp.zeros_like(bes_ref)
    scal = scal_ref[...]
    nb = scal.shape[0]
    valid = scal[:, 0:1]
    pe = jnp.sum(qpe0_ref[...] * phi1_ref[...], axis=1, keepdims=True)
    nev = jnp.sum(qnui_ref[...] * phiuj_ref[...], axis=1, keepdims=True) * valid
    ones = jnp.ones((nb, 1), jnp.float32)
    zcol = jnp.zeros((nb, 1), jnp.float32)
    iota_g = lax.broadcasted_iota(jnp.int32, (nb, G), 1)
    iota_k = lax.broadcasted_iota(jnp.int32, (nb, KP), 1)
    gebt = jnp.where(scal[:, 2:3].astype(jnp.int32) == iota_g, 1.0, 0.0)
    ggit = jnp.where(scal[:, 1:2].astype(jnp.int32) == iota_g, 1.0, 0.0)
    k1h = jnp.where(scal[:, 3:4].astype(jnp.int32) == iota_k, 1.0, 0.0)
    zw = scal[:, 6:7] * scal[:, 4:5]
    bv = zw * jnp.exp(scal[:, 5:6])
    s6a = jnp.concatenate([pe, ones, zcol, zcol, zcol, zcol, zcol, zcol], axis=1)
    s6b = jnp.concatenate([zcol, zcol, nev, valid, zcol, zcol, zcol, zcol],
                          axis=1)
    out1_ref[...] += (
        lax.dot_general(gebt, s6a, (((0,), (0,)), ((), ())),
                        preferred_element_type=jnp.float32)
        + lax.dot_general(ggit, s6b, (((0,), (0,)), ((), ())),
                          preferred_element_type=jnp.float32))
    zew_ref[...] += lax.dot_general(gebt, k1h * zw, (((0,), (0,)), ((), ())),
                                    preferred_element_type=jnp.float32)
    bes_ref[...] += lax.dot_general(gebt, k1h * bv, (((0,), (0,)), ((), ())),
                                    preferred_element_type=jnp.float32)


def _tc_i(out1_ref, sega_ref, segb_ref, kl1_ref, saux_ref, zew_ref, bes_ref,
          mze_ref, out_ref):
    out1 = out1_ref[...]
    p_edge = out1[:, 0]
    np_ = out1[:, 1]
    necorr = out1[:, 2]
    cnt = out1[:, 3]
    kl2 = sega_ref[:, 0]
    recon_word = -sega_ref[:, 1]
    n1g = segb_ref[:, 0]
    kl1 = kl1_ref[:, 0]
    sizes = saux_ref[:, 0]
    n_edge1 = n1g - necorr
    nn = sizes * (sizes - 1.0) - cnt
    recon_structure = -(p_edge + n_edge1 / (nn + 1e-6) * np_)
    logze = mze_ref[2:3, :]
    bes = bes_ref[...] * jnp.exp(-logze)
    lr = (jnp.log(jnp.clip(bes, 1e-10, None))
          - jnp.log(jnp.clip(zew_ref[...], 1e-10, None)))
    recon_edge = -jnp.sum(lr, axis=1)
    loss = recon_edge + recon_word + kl1 + kl2 + recon_structure
    vals = [loss, recon_word, recon_edge, recon_structure, p_edge, kl1, kl2]
    r = lax.broadcasted_iota(jnp.int32, (8, NI), 0)
    c = lax.broadcasted_iota(jnp.int32, (8, NI), 1)
    acc = jnp.zeros((8, NI), jnp.float32)
    for i, v in enumerate(vals):
        mv = jnp.sum(v) / G
        acc = acc + jnp.where(jnp.logical_and(r == i, c == 0), mv, 0.0)
    out_ref[...] = acc


def _f32(x):
    return jnp.asarray(x, jnp.float32)


def kernel(idx_x, x_batch, idx_w, edge_w, edge_id, edge_id_batch, edge_index,
           whole_edge, word_vec, word_vec_beta, topic_vec, topic_edge_vec,
           W_topic, W_enc, b_enc, W_phi, b_phi, W_mu, b_mu, W_lv, b_lv):
    key = jax.random.key(42)
    idx_x = idx_x.astype(jnp.int32)
    x_batch = x_batch.astype(jnp.int32)
    edge_id = edge_id.astype(jnp.int32)
    edge_id_batch = edge_id_batch.astype(jnp.int32)
    e0 = edge_index[0].astype(jnp.int32)
    e1 = edge_index[1].astype(jnp.int32)
    we0 = whole_edge[0].astype(jnp.int32)
    we1 = whole_edge[1].astype(jnp.int32)
    word_vec = _f32(word_vec)
    wvb = _f32(word_vec_beta)

    skey = jnp.sort(e0 * N + e1)
    uniq = jnp.concatenate(
        [jnp.ones((1,), jnp.float32),
         (skey[1:] != skey[:-1]).astype(jnp.float32)])

    tvcat = jnp.zeros((KP, 3 * NI), jnp.float32).at[:K].set(
        jnp.concatenate([_f32(topic_vec), _f32(topic_edge_vec)], axis=-1))
    tvp = jnp.zeros((KP, NI), jnp.float32).at[:K].set(_f32(topic_vec))
    t1p = jnp.zeros((KP, NI), jnp.float32).at[:K].set(_f32(topic_edge_vec[:, :NI]))
    t2p = jnp.zeros((KP, NI), jnp.float32).at[:K].set(_f32(topic_edge_vec[:, NI:]))
    wphi_p = jnp.zeros((NI, KP), jnp.float32).at[:, :K].set(_f32(W_phi))
    bphi_p = jnp.full((1, KP), -1e30, jnp.float32).at[0, :K].set(_f32(b_phi))
    wmu_p = jnp.zeros((NI, KP), jnp.float32).at[:, :K].set(_f32(W_mu))
    bmu_p = jnp.zeros((1, KP), jnp.float32).at[0, :K].set(_f32(b_mu))
    wlv_p = jnp.zeros((NI, KP), jnp.float32).at[:, :K].set(_f32(W_lv))
    blv_p = jnp.zeros((1, KP), jnp.float32).at[0, :K].set(_f32(b_lv))
    benc = _f32(b_enc).reshape(1, NI)
    wvb_p = jnp.zeros((VP, NI), jnp.float32).at[:V].set(wvb)
    we0p = jnp.zeros((EWP,), jnp.int32).at[:EW].set(we0)
    we1p = jnp.zeros((EWP,), jnp.int32).at[:EW].set(we1)

    eps = jax.random.normal(jax.random.fold_in(key, 1), (G, K))
    eps_p = jnp.zeros((G, KP), jnp.float32).at[:, :K].set(eps)
    u = jax.random.uniform(jax.random.fold_in(key, 2), (1, N, K),
                           minval=1e-10, maxval=1.0)
    gumb = -jnp.log(-jnp.log(u))[0]
    gumb_p = jnp.full((N, KP), -1e30, jnp.float32).at[:, :K].set(gumb)

    sc1, sc2, sc3 = _sc_kernels()
    zinit = jnp.zeros((N, NI), jnp.float32)
    h0, h0b, msg2, ui, uj, gi, val0 = sc1(word_vec, wvb, idx_x, e0, e1,
                                          _f32(edge_w), zinit, skey, x_batch)
    valid = val0 * uniq

    NB = 512
    h, phi, qp, qn, kz3 = pl.pallas_call(
        _tc_c,
        grid=(N // NB,),
        in_specs=[
            pl.BlockSpec((NB, NI), lambda i: (i, 0)),
            pl.BlockSpec((NB, NI), lambda i: (i, 0)),
            pl.BlockSpec((NB, NI), lambda i: (i, 0)),
            pl.BlockSpec((NI, NI), lambda i: (0, 0)),
            pl.BlockSpec((1, NI), lambda i: (0, 0)),
            pl.BlockSpec((NI, KP), lambda i: (0, 0)),
            pl.BlockSpec((1, KP), lambda i: (0, 0)),
            pl.BlockSpec((KP, 3 * NI), lambda i: (0, 0)),
            pl.BlockSpec((G, 3 * NI), lambda i: (0, 0)),
            pl.BlockSpec((NB, KP), lambda i: (i, 0)),
        ],
        out_specs=[
            pl.BlockSpec((NB, NI), lambda i: (i, 0)),
            pl.BlockSpec((NB, KP), lambda i: (i, 0)),
            pl.BlockSpec((NB, KP), lambda i: (i, 0)),
            pl.BlockSpec((NB, KP), lambda i: (i, 0)),
            pl.BlockSpec((1, 1, NB), lambda i: (i, 0, 0)),
        ],
        out_shape=[
            jax.ShapeDtypeStruct((N, NI), jnp.float32),
            jax.ShapeDtypeStruct((N, KP), jnp.float32),
            jax.ShapeDtypeStruct((N, KP), jnp.float32),
            jax.ShapeDtypeStruct((N, KP), jnp.float32),
            jax.ShapeDtypeStruct((N // NB, 1, NB), jnp.int32),
        ],
    )(h0, msg2[0], msg2[1], _f32(W_enc), benc, wphi_p, bphi_p, tvcat,
      _f32(W_topic), gumb_p)
    kz = kz3.reshape(N)

    xb_3d = x_batch.reshape(N // NB, 1, NB)
    iw_3d = _f32(idx_w).reshape(N // NB, 1, NB)
    gnum, waux, saux, pg = pl.pallas_call(
        _tc_d1,
        grid=(N // NB,),
        in_specs=[
            pl.BlockSpec((1, 1, NB), lambda i: (i, 0, 0)),
            pl.BlockSpec((1, 1, NB), lambda i: (i, 0, 0)),
            pl.BlockSpec((NB, NI), lambda i: (i, 0)),
            pl.BlockSpec((NB, KP), lambda i: (i, 0)),
        ],
        out_specs=[
            pl.BlockSpec((G, NI), lambda i: (0, 0)),
            pl.BlockSpec((G, NI), lambda i: (0, 0)),
            pl.BlockSpec((G, NI), lambda i: (0, 0)),
            pl.BlockSpec((G, KP), lambda i: (0, 0)),
        ],
        out_shape=[
            jax.ShapeDtypeStruct((G, NI), jnp.float32),
            jax.ShapeDtypeStruct((G, NI), jnp.float32),
            jax.ShapeDtypeStruct((G, NI), jnp.float32),
            jax.ShapeDtypeStruct((G, KP), jnp.float32),
        ],
    )(xb_3d, iw_3d, h, phi)

    VB = 512
    at, bt, mzb = pl.pallas_call(
        _tc_f,
        grid=(VP // VB,),
        in_specs=[
            pl.BlockSpec((VB, NI), lambda i: (i, 0)),
            pl.BlockSpec((KP, NI), lambda i: (0, 0)),
            pl.BlockSpec((KP, NI), lambda i: (0, 0)),
            pl.BlockSpec((KP, NI), lambda i: (0, 0)),
        ],
        out_specs=[
            pl.BlockSpec((VB, KP), lambda i: (i, 0)),
            pl.BlockSpec((VB, KP), lambda i: (i, 0)),
            pl.BlockSpec((8, KP), lambda i: (0, 0)),
        ],
        out_shape=[
            jax.ShapeDtypeStruct((VP, KP), jnp.float32),
            jax.ShapeDtypeStruct((VP, KP), jnp.float32),
            jax.ShapeDtypeStruct((8, KP), jnp.float32),
        ],
    )(wvb_p, tvp, t1p, t2p)

    sega, segb, kl1b = pl.pallas_call(
        _tc_e,
        grid=(N // NB,),
        in_specs=[
            pl.BlockSpec((1, 1, NB), lambda i: (i, 0, 0)),
            pl.BlockSpec((1, 1, NB), lambda i: (i, 0, 0)),
            pl.BlockSpec((NB, KP), lambda i: (i, 0)),
            pl.BlockSpec((NB, KP), lambda i: (i, 0)),
            pl.BlockSpec((NB, NI), lambda i: (i, 0)),
            pl.BlockSpec((KP, NI), lambda i: (0, 0)),
            pl.BlockSpec((G, NI), lambda i: (0, 0)),
            pl.BlockSpec((G, NI), lambda i: (0, 0)),
            pl.BlockSpec((NI, KP), lambda i: (0, 0)),
            pl.BlockSpec((1, KP), lambda i: (0, 0)),
            pl.BlockSpec((NI, KP), lambda i: (0, 0)),
            pl.BlockSpec((1, KP), lambda i: (0, 0)),
            pl.BlockSpec((G, KP), lambda i: (0, 0)),
            pl.BlockSpec((G, KP), lambda i: (0, 0)),
            pl.BlockSpec((8, KP), lambda i: (0, 0)),
        ],
        out_specs=[
            pl.BlockSpec((G, 8), lambda i: (0, 0)),
            pl.BlockSpec((G, 8), lambda i: (0, 0)),
            pl.BlockSpec((G, 8), lambda i: (0, 0)),
        ],
        out_shape=[
            jax.ShapeDtypeStruct((G, 8), jnp.float32),
            jax.ShapeDtypeStruct((G, 8), jnp.float32),
            jax.ShapeDtypeStruct((G, 8), jnp.float32),
        ],
    )(xb_3d, iw_3d, phi, qn, h0b, tvp, gnum, waux, wmu_p, bmu_p, wlv_p,
      blv_p, eps_p, pg, mzb)

    phi1, qpe0, qnui, phiuj, sab = sc2(
        phi, qp, qn, at, bt, e0, e1, ui, uj, we0p, we1p)

    WB = 512
    mze = pl.pallas_call(
        _tc_g,
        grid=(EWP // WB,),
        in_specs=[
            pl.BlockSpec((WB, KP), lambda i: (i, 0)),
        ],
        out_specs=pl.BlockSpec((8, KP), lambda i: (0, 0)),
        out_shape=jax.ShapeDtypeStruct((8, KP), jnp.float32),
    )(sab)

    sval, kz0, same = sc3(sab, edge_id, kz, e0, e1)

    scal = jnp.stack(
        [valid, gi.astype(jnp.float32), edge_id_batch.astype(jnp.float32),
         kz0.astype(jnp.float32), same, sval, _f32(edge_w),
         jnp.zeros((E,), jnp.float32)], axis=1)

    EB2 = 512
    out1, zew, bes = pl.pallas_call(
        _tc_h,
        grid=(E // EB2,),
        in_specs=[
            pl.BlockSpec((EB2, KP), lambda i: (i, 0)),
            pl.BlockSpec((EB2, KP), lambda i: (i, 0)),
            pl.BlockSpec((EB2, KP), lambda i: (i, 0)),
            pl.BlockSpec((EB2, KP), lambda i: (i, 0)),
            pl.BlockSpec((EB2, 8), lambda i: (i, 0)),
        ],
        out_specs=[
            pl.BlockSpec((G, 8), lambda i: (0, 0)),
            pl.BlockSpec((G, KP), lambda i: (0, 0)),
            pl.BlockSpec((G, KP), lambda i: (0, 0)),
        ],
        out_shape=[
            jax.ShapeDtypeStruct((G, 8), jnp.float32),
            jax.ShapeDtypeStruct((G, KP), jnp.float32),
            jax.ShapeDtypeStruct((G, KP), jnp.float32),
        ],
    )(qpe0, phi1, qnui, phiuj, scal)

    out = pl.pallas_call(
        _tc_i,
        out_shape=jax.ShapeDtypeStruct((8, NI), jnp.float32),
    )(out1, sega, segb, kl1b, saux, zew, bes, mze)

    return (out[0, 0], out[1, 0], out[2, 0], out[3, 0], out[4, 0],
            out[5, 0], out[6, 0])
